# baseline (speedup 1.0000x reference)
import functools

import jax
import jax.numpy as jnp
from jax import lax
from jax.experimental import pallas as pl
from jax.experimental.pallas import tpu as pltpu
from jax.experimental.pallas import tpu_sc as plsc

N = 10000
E = 160000
EMB = 64
LOUT = 4
ODIM = 2
HPAD = 16

NC = 2
NS = 16
NW = NC * NS
CH = E // NW
CHG = 1000
RPS = N // NS

BLK = 6400
NBLK = E // BLK
PB = BLK // 8

def _sc_mesh():
    return plsc.VectorSubcoreMesh(core_axis_name="c", subcore_axis_name="s",
                                  num_cores=NC, num_subcores=NS)


@jax.jit
def _sc_gather(h16, src, dst):
    @functools.partial(
        pl.kernel,
        out_type=(jax.ShapeDtypeStruct((E, HPAD), jnp.float32),
                  jax.ShapeDtypeStruct((E, HPAD), jnp.float32)),
        mesh=_sc_mesh(),
        compiler_params=pltpu.CompilerParams(use_tc_tiling_on_sc=False),
        scratch_types=[
            pltpu.VMEM((CH // CHG, CHG), jnp.int32),
            pltpu.VMEM((CH // CHG, CHG), jnp.int32),
            pltpu.VMEM((CHG, HPAD), jnp.float32),
            pltpu.VMEM((CHG, HPAD), jnp.float32),
            pltpu.SemaphoreType.DMA,
            pltpu.SemaphoreType.DMA,
            pltpu.SemaphoreType.DMA,
        ],
    )
    def k(h_hbm, src_hbm, dst_hbm, gdst_hbm, gsrc_hbm,
          idx_d, idx_s, buf0, buf1, sem_i, sem_g, sem_w):
        wid = lax.axis_index("s") * NC + lax.axis_index("c")
        base = wid * CH
        cpi_d = pltpu.async_copy(dst_hbm.at[wid], idx_d, sem_i)
        cpi_s = pltpu.async_copy(src_hbm.at[wid], idx_s, sem_i)
        cpi_d.wait()
        cpi_s.wait()

        nch = CH // CHG
        bufs = (buf0, buf1)
        outs = []
        for j in range(2 * nch):
            idx = idx_d if j < nch else idx_s
            ohbm = gdst_hbm if j < nch else gsrc_hbm
            c = (j % nch) * CHG
            buf = bufs[j % 2]
            if j >= 2:
                outs[j - 2].wait()
            pltpu.async_copy(h_hbm.at[idx.at[j % nch]], buf, sem_g).wait()
            outs.append(
                pltpu.async_copy(buf, ohbm.at[pl.ds(base + c, CHG)], sem_w))
        outs[-2].wait()
        outs[-1].wait()

    return k(h16, src.reshape(NW, CH // CHG, CHG),
             dst.reshape(NW, CH // CHG, CHG))


@jax.jit
def _sc_scatter(payload, dst, zrows):
    @functools.partial(
        pl.kernel,
        out_type=jax.ShapeDtypeStruct((NC, N, HPAD), jnp.float32),
        mesh=_sc_mesh(),
        compiler_params=pltpu.CompilerParams(use_tc_tiling_on_sc=False),
        scratch_types=[
            pltpu.VMEM((CH // CHG, CHG), jnp.int32),
            pltpu.VMEM((CHG, HPAD), jnp.float32),
            pltpu.VMEM((CHG, HPAD), jnp.float32),
            pltpu.VMEM_SHARED((N, HPAD), jnp.float32),
            pltpu.SemaphoreType.DMA,
            pltpu.SemaphoreType.DMA,
        ],
    )
    def k(pay_hbm, dst_hbm, z_hbm, out_hbm, idx_v, buf0, buf1, acc,
          sem_i, sem_p):
        c = lax.axis_index("c")
        s = lax.axis_index("s")
        wid = s * NC + c
        base = wid * CH
        cpz = pltpu.async_copy(z_hbm, acc.at[pl.ds(s * RPS, RPS)], sem_i)
        cpi = pltpu.async_copy(dst_hbm.at[wid], idx_v, sem_i)
        nch = CH // CHG
        bufs = (buf0, buf1)
        loads = [pltpu.async_copy(pay_hbm.at[pl.ds(base + j * CHG, CHG)],
                                  bufs[j % 2], sem_p) for j in range(2)]
        cpz.wait()
        cpi.wait()
        plsc.subcore_barrier()
        for j in range(nch):
            loads[j].wait()
            pltpu.sync_copy(bufs[j % 2], acc.at[idx_v.at[j]], add=True)
            if j + 2 < nch:
                loads.append(
                    pltpu.async_copy(pay_hbm.at[pl.ds(base + (j + 2) * CHG,
                                                      CHG)],
                                     bufs[j % 2], sem_p))
        plsc.subcore_barrier()
        pltpu.sync_copy(acc.at[pl.ds(s * RPS, RPS)],
                        out_hbm.at[c].at[pl.ds(s * RPS, RPS)])

    return k(payload, dst.reshape(NW, CH // CHG, CHG), zrows)


def _unpack(xp):
    t = jnp.transpose(xp)
    return jnp.concatenate(
        [t[HPAD * g:HPAD * (g + 1), :] for g in range(8)], axis=1)


def _pack(x):
    t = jnp.concatenate(
        [x[:, PB * g:PB * (g + 1)] for g in range(8)], axis=0)
    return jnp.transpose(t)


def _edge_mlp_body(gdst, gsrc, w0t, w1t, w2t, w3t, auxt, b3t, out_ref,
                   act, stats):
    p = pl.program_id(0)
    i = pl.program_id(1)
    blk = pl.ds(i * BLK, BLK)
    inv_e = 1.0 / E

    @pl.when((p == 0) & (i == 0))
    def _():
        stats[...] = jnp.zeros_like(stats)

    def bn_tanh(x, li, g_col, h_col):
        m = stats[:, 2 * li:2 * li + 1] * inv_e
        v = stats[:, 2 * li + 1:2 * li + 2] * inv_e - m * m
        a = auxt[:, g_col:g_col + 1] * jax.lax.rsqrt(v + 1e-5)
        cc = auxt[:, h_col:h_col + 1] - m * a
        return jnp.tanh(x * a + cc)

    def put_stats(li, x):
        stats[:, 2 * li:2 * li + 1] += jnp.sum(x, axis=1, keepdims=True)
        stats[:, 2 * li + 1:2 * li + 2] += jnp.sum(x * x, axis=1,
                                                   keepdims=True)

    @pl.when(p == 0)
    def _():
        hdt = _unpack(gdst[...] - gsrc[...])
        x1 = jnp.dot(w0t[...], hdt,
                     preferred_element_type=jnp.float32) + auxt[:, 0:1]
        act[:, blk] = x1
        put_stats(0, x1)

    @pl.when(p == 1)
    def _():
        t1 = bn_tanh(act[:, blk], 0, 1, 2)
        x2 = jnp.dot(w1t[...], t1,
                     preferred_element_type=jnp.float32) + auxt[:, 3:4]
        act[:, blk] = x2
        put_stats(1, x2)

    @pl.when(p == 2)
    def _():
        t2 = bn_tanh(act[:, blk], 1, 4, 5)
        x3 = jnp.dot(w2t[...], t2,
                     preferred_element_type=jnp.float32) + auxt[:, 6:7]
        act[:, blk] = x3
        put_stats(2, x3)

    @pl.when(p == 3)
    def _():
        t3 = bn_tanh(act[:, blk], 2, 7, 8)
        pay = jnp.dot(w3t[...], t3,
                      preferred_element_type=jnp.float32) + b3t[...]
        hdt = _unpack(gdst[...] - gsrc[...])
        mask = jnp.all(hdt == 0.0, axis=0, keepdims=True)
        row = lax.broadcasted_iota(jnp.int32, (HPAD, BLK), 0)
        pay = jnp.where(mask & (row < LOUT), 0.0, pay)
        out_ref[...] = _pack(pay)


@jax.jit
def _tc_edge_mlp(gdst_p, gsrc_p, w0t, w1t, w2t, w3t, auxt, b3t):
    edge_map = lambda p, i: (jnp.where((p == 0) | (p == 3), i, 0), 0)
    rep = lambda p, i: (0, 0)
    return pl.pallas_call(
        _edge_mlp_body,
        grid=(4, NBLK),
        in_specs=[
            pl.BlockSpec((PB, 128), edge_map),
            pl.BlockSpec((PB, 128), edge_map),
            pl.BlockSpec((EMB, HPAD), rep),
            pl.BlockSpec((EMB, EMB), rep),
            pl.BlockSpec((EMB, EMB), rep),
            pl.BlockSpec((HPAD, EMB), rep),
            pl.BlockSpec((EMB, 9), rep),
            pl.BlockSpec((HPAD, 1), rep),
        ],
        out_specs=pl.BlockSpec((PB, 128),
                               lambda p, i: (jnp.where(p == 3, i, 0), 0)),
        out_shape=jax.ShapeDtypeStruct((E // 8, 128), jnp.float32),
        scratch_shapes=[
            pltpu.VMEM((EMB, E), jnp.float32),
            pltpu.VMEM((EMB, 8), jnp.float32),
        ],
    )(gdst_p, gsrc_p, w0t, w1t, w2t, w3t, auxt, b3t)


def _node_mlp_body(parts, wu0, wu1, wu2, wu3, auxu, tail, wp, out_ref):
    s = parts[0] + parts[1]
    cnt = jnp.maximum(s[:, LOUT:LOUT + 1], 1.0)
    aggr = jnp.concatenate(
        [s[:, 0:2], s[:, 2:4] / cnt,
         jnp.zeros((N, HPAD - LOUT), jnp.float32)], axis=1)

    def bn_tanh(x, g, h):
        m = jnp.mean(x, axis=0, keepdims=True)
        v = jnp.mean((x - m) * (x - m), axis=0, keepdims=True)
        return jnp.tanh(g * (x - m) * jax.lax.rsqrt(v + 1e-5) + h)

    x = jnp.dot(aggr, wu0[...],
                preferred_element_type=jnp.float32) + auxu[0:1, :]
    x = bn_tanh(x, auxu[1:2, :], auxu[2:3, :])
    x = jnp.dot(x, wu1[...],
                preferred_element_type=jnp.float32) + auxu[3:4, :]
    x = bn_tanh(x, auxu[4:5, :], auxu[5:6, :])
    x = jnp.dot(x, wu2[...],
                preferred_element_type=jnp.float32) + auxu[6:7, :]
    x = bn_tanh(x, auxu[7:8, :], auxu[8:9, :])
    x = jnp.dot(x, wu3[...],
                preferred_element_type=jnp.float32) + tail[0:1, :]
    x = bn_tanh(x, tail[1:2, :], tail[2:3, :])
    out_ref[...] = jnp.dot(x, wp[...],
                           preferred_element_type=jnp.float32) + tail[3:4, 0:ODIM]


@jax.jit
def _tc_node_mlp(parts, wu0, wu1, wu2, wu3, auxu, tail, wp):
    return pl.pallas_call(
        _node_mlp_body,
        out_shape=jax.ShapeDtypeStruct((N, ODIM), jnp.float32),
    )(parts, wu0, wu1, wu2, wu3, auxu, tail, wp)


def kernel(pos, vel, edge_index, params):
    f32 = jnp.float32
    h16 = jnp.concatenate(
        [pos, vel, jnp.zeros((N, HPAD - 4), f32)], axis=1)
    src = edge_index[0]
    dst = edge_index[1]

    w0t = jnp.concatenate(
        [params['Wm0'], jnp.zeros((HPAD - 4, EMB), f32)], axis=0).T
    w3t = jnp.concatenate(
        [params['Wm3'], jnp.zeros((EMB, HPAD - LOUT), f32)], axis=1).T
    b3t = jnp.concatenate(
        [params['bm3'], jnp.ones((1,), f32),
         jnp.zeros((HPAD - LOUT - 1,), f32)], axis=0).reshape(HPAD, 1)
    auxt = jnp.stack([
        params['bm0'], params['gm1'], params['hm1'],
        params['bm1'], params['gm2'], params['hm2'],
        params['bm2'], params['gm3'], params['hm3'],
    ], axis=1)

    wu0 = jnp.concatenate(
        [params['Wu0'], jnp.zeros((HPAD - LOUT, EMB), f32)], axis=0)
    auxu = jnp.stack([
        params['bu0'], params['gu1'], params['hu1'],
        params['bu1'], params['gu2'], params['hu2'],
        params['bu2'], params['gu3'], params['hu3'],
    ], axis=0)
    tail = jnp.stack([
        params['bu3'], params['gu4'], params['hu4'],
        jnp.concatenate([params['bp'], jnp.zeros((LOUT - ODIM,), f32)]),
    ], axis=0)

    gdst, gsrc = _sc_gather(h16, src, dst)
    payload = _tc_edge_mlp(gdst.reshape(E // 8, 128), gsrc.reshape(E // 8, 128),
                           w0t, params['Wm1'].T, params['Wm2'].T,
                           w3t, auxt, b3t).reshape(E, HPAD)
    zrows = jnp.zeros((RPS, HPAD), f32)
    parts = _sc_scatter(payload, dst, zrows)
    out = _tc_node_mlp(parts, wu0, params['Wu1'], params['Wu2'],
                       params['Wu3'], auxu, tail, params['Wp'])
    return out


# speedup vs baseline: 1.0398x; 1.0398x over previous
import functools

import jax
import jax.numpy as jnp
from jax import lax
from jax.experimental import pallas as pl
from jax.experimental.pallas import tpu as pltpu
from jax.experimental.pallas import tpu_sc as plsc

N = 10000
E = 160000
EMB = 64
LOUT = 4
ODIM = 2
HPAD = 16

NC = 2
NS = 16
NW = NC * NS
CH = E // NW
CHG = 1000
RPS = N // NS

BLK = 6400
NBLK = E // BLK
PB = BLK // 8

def _sc_mesh():
    return plsc.VectorSubcoreMesh(core_axis_name="c", subcore_axis_name="s",
                                  num_cores=NC, num_subcores=NS)


@jax.jit
def _sc_gather(h16, src, dst):
    @functools.partial(
        pl.kernel,
        out_type=(jax.ShapeDtypeStruct((E, HPAD), jnp.float32),
                  jax.ShapeDtypeStruct((E, HPAD), jnp.float32)),
        mesh=_sc_mesh(),
        compiler_params=pltpu.CompilerParams(use_tc_tiling_on_sc=False),
        scratch_types=[
            pltpu.VMEM((CH // CHG, CHG), jnp.int32),
            pltpu.VMEM((CH // CHG, CHG), jnp.int32),
            pltpu.VMEM((CHG, HPAD), jnp.float32),
            pltpu.VMEM((CHG, HPAD), jnp.float32),
            pltpu.SemaphoreType.DMA,
            pltpu.SemaphoreType.DMA,
            pltpu.SemaphoreType.DMA,
        ],
    )
    def k(h_hbm, src_hbm, dst_hbm, gdst_hbm, gsrc_hbm,
          idx_d, idx_s, buf0, buf1, sem_i, sem_g, sem_w):
        wid = lax.axis_index("s") * NC + lax.axis_index("c")
        base = wid * CH
        cpi_d = pltpu.async_copy(dst_hbm.at[wid], idx_d, sem_i)
        cpi_s = pltpu.async_copy(src_hbm.at[wid], idx_s, sem_i)
        cpi_d.wait()
        cpi_s.wait()

        nch = CH // CHG
        bufs = (buf0, buf1)
        outs = []
        for j in range(2 * nch):
            idx = idx_d if j < nch else idx_s
            ohbm = gdst_hbm if j < nch else gsrc_hbm
            c = (j % nch) * CHG
            buf = bufs[j % 2]
            if j >= 2:
                outs[j - 2].wait()
            pltpu.async_copy(h_hbm.at[idx.at[j % nch]], buf, sem_g).wait()
            outs.append(
                pltpu.async_copy(buf, ohbm.at[pl.ds(base + c, CHG)], sem_w))
        outs[-2].wait()
        outs[-1].wait()

    return k(h16, src.reshape(NW, CH // CHG, CHG),
             dst.reshape(NW, CH // CHG, CHG))


@jax.jit
def _sc_scatter(payload, dst, zrows):
    @functools.partial(
        pl.kernel,
        out_type=jax.ShapeDtypeStruct((NC, N, HPAD), jnp.float32),
        mesh=_sc_mesh(),
        compiler_params=pltpu.CompilerParams(use_tc_tiling_on_sc=False),
        scratch_types=[
            pltpu.VMEM((CH // CHG, CHG), jnp.int32),
            pltpu.VMEM((CHG, HPAD), jnp.float32),
            pltpu.VMEM((CHG, HPAD), jnp.float32),
            pltpu.VMEM_SHARED((N, HPAD), jnp.float32),
            pltpu.SemaphoreType.DMA,
            pltpu.SemaphoreType.DMA,
        ],
    )
    def k(pay_hbm, dst_hbm, z_hbm, out_hbm, idx_v, buf0, buf1, acc,
          sem_i, sem_p):
        c = lax.axis_index("c")
        s = lax.axis_index("s")
        wid = s * NC + c
        base = wid * CH
        cpz = pltpu.async_copy(z_hbm, acc.at[pl.ds(s * RPS, RPS)], sem_i)
        cpi = pltpu.async_copy(dst_hbm.at[wid], idx_v, sem_i)
        nch = CH // CHG
        bufs = (buf0, buf1)
        loads = [pltpu.async_copy(pay_hbm.at[pl.ds(base + j * CHG, CHG)],
                                  bufs[j % 2], sem_p) for j in range(2)]
        cpz.wait()
        cpi.wait()
        plsc.subcore_barrier()
        for j in range(nch):
            loads[j].wait()
            pltpu.sync_copy(bufs[j % 2], acc.at[idx_v.at[j]], add=True)
            if j + 2 < nch:
                loads.append(
                    pltpu.async_copy(pay_hbm.at[pl.ds(base + (j + 2) * CHG,
                                                      CHG)],
                                     bufs[j % 2], sem_p))
        plsc.subcore_barrier()
        pltpu.sync_copy(acc.at[pl.ds(s * RPS, RPS)],
                        out_hbm.at[c].at[pl.ds(s * RPS, RPS)])

    return k(payload, dst.reshape(NW, CH // CHG, CHG), zrows)


def _unpack(xp):
    t = jnp.transpose(xp)
    return jnp.concatenate(
        [t[HPAD * g:HPAD * (g + 1), :] for g in range(8)], axis=1)


def _pack(x):
    t = jnp.concatenate(
        [x[:, PB * g:PB * (g + 1)] for g in range(8)], axis=0)
    return jnp.transpose(t)


def _mm(w, x):
    return lax.dot_general(w, x, (((0,), (0,)), ((), ())),
                           preferred_element_type=jnp.float32)


def _edge_mlp_body(gdst, gsrc, w0, w1, w2, w3, auxt, out_ref,
                   act, stats):
    p = pl.program_id(0)
    i = pl.program_id(1)
    blk = pl.ds(i * BLK, BLK)
    inv_e = 1.0 / E

    @pl.when((p == 0) & (i == 0))
    def _():
        stats[...] = jnp.zeros_like(stats)

    def bn_tanh(x, li, g_col, h_col):
        m = stats[:, 2 * li:2 * li + 1] * inv_e
        v = stats[:, 2 * li + 1:2 * li + 2] * inv_e - m * m
        a = auxt[:, g_col:g_col + 1] * jax.lax.rsqrt(v + 1e-5)
        cc = auxt[:, h_col:h_col + 1] - m * a
        return jnp.tanh(x * a + cc)

    def put_stats(li, x):
        stats[:, 2 * li:2 * li + 1] += jnp.sum(x, axis=1, keepdims=True)
        stats[:, 2 * li + 1:2 * li + 2] += jnp.sum(x * x, axis=1,
                                                   keepdims=True)

    @pl.when(p == 0)
    def _():
        hdt = _unpack(gdst[...] - gsrc[...])
        x1 = _mm(w0[...], hdt[0:4, :]) + auxt[:, 0:1]
        act[:, blk] = x1
        put_stats(0, x1)

    @pl.when(p == 1)
    def _():
        t1 = bn_tanh(act[:, blk], 0, 1, 2)
        x2 = _mm(w1[...], t1) + auxt[:, 3:4]
        act[:, blk] = x2
        put_stats(1, x2)

    @pl.when(p == 2)
    def _():
        t2 = bn_tanh(act[:, blk], 1, 4, 5)
        x3 = _mm(w2[...], t2) + auxt[:, 6:7]
        act[:, blk] = x3
        put_stats(2, x3)

    @pl.when(p == 3)
    def _():
        t3 = bn_tanh(act[:, blk], 2, 7, 8)
        pay4 = _mm(w3[...], t3) + auxt[0:4, 9:10]
        hdt = _unpack(gdst[...] - gsrc[...])
        mask = jnp.all(hdt == 0.0, axis=0, keepdims=True)
        pay4 = jnp.where(mask, 0.0, pay4)
        pay = jnp.concatenate(
            [pay4, jnp.ones((1, BLK), jnp.float32),
             jnp.zeros((HPAD - LOUT - 1, BLK), jnp.float32)], axis=0)
        out_ref[...] = _pack(pay)


@jax.jit
def _tc_edge_mlp(gdst_p, gsrc_p, w0, w1, w2, w3, auxt):
    edge_map = lambda p, i: (jnp.where((p == 0) | (p == 3), i, 0), 0)
    rep = lambda p, i: (0, 0)
    return pl.pallas_call(
        _edge_mlp_body,
        grid=(4, NBLK),
        in_specs=[
            pl.BlockSpec((PB, 128), edge_map),
            pl.BlockSpec((PB, 128), edge_map),
            pl.BlockSpec((4, EMB), rep),
            pl.BlockSpec((EMB, EMB), rep),
            pl.BlockSpec((EMB, EMB), rep),
            pl.BlockSpec((EMB, LOUT), rep),
            pl.BlockSpec((EMB, 10), rep),
        ],
        out_specs=pl.BlockSpec((PB, 128),
                               lambda p, i: (jnp.where(p == 3, i, 0), 0)),
        out_shape=jax.ShapeDtypeStruct((E // 8, 128), jnp.float32),
        scratch_shapes=[
            pltpu.VMEM((EMB, E), jnp.float32),
            pltpu.VMEM((EMB, 8), jnp.float32),
        ],
    )(gdst_p, gsrc_p, w0, w1, w2, w3, auxt)


def _node_mlp_body(parts, wu0, wu1, wu2, wu3, auxu, tail, wp, out_ref):
    s = parts[0] + parts[1]
    cnt = jnp.maximum(s[:, LOUT:LOUT + 1], 1.0)
    aggr = jnp.concatenate(
        [s[:, 0:2], s[:, 2:4] / cnt,
         jnp.zeros((N, HPAD - LOUT), jnp.float32)], axis=1)

    def bn_tanh(x, g, h):
        m = jnp.mean(x, axis=0, keepdims=True)
        v = jnp.mean((x - m) * (x - m), axis=0, keepdims=True)
        return jnp.tanh(g * (x - m) * jax.lax.rsqrt(v + 1e-5) + h)

    x = jnp.dot(aggr, wu0[...],
                preferred_element_type=jnp.float32) + auxu[0:1, :]
    x = bn_tanh(x, auxu[1:2, :], auxu[2:3, :])
    x = jnp.dot(x, wu1[...],
                preferred_element_type=jnp.float32) + auxu[3:4, :]
    x = bn_tanh(x, auxu[4:5, :], auxu[5:6, :])
    x = jnp.dot(x, wu2[...],
                preferred_element_type=jnp.float32) + auxu[6:7, :]
    x = bn_tanh(x, auxu[7:8, :], auxu[8:9, :])
    x = jnp.dot(x, wu3[...],
                preferred_element_type=jnp.float32) + tail[0:1, :]
    x = bn_tanh(x, tail[1:2, :], tail[2:3, :])
    out_ref[...] = jnp.dot(x, wp[...],
                           preferred_element_type=jnp.float32) + tail[3:4, 0:ODIM]


@jax.jit
def _tc_node_mlp(parts, wu0, wu1, wu2, wu3, auxu, tail, wp):
    return pl.pallas_call(
        _node_mlp_body,
        out_shape=jax.ShapeDtypeStruct((N, ODIM), jnp.float32),
    )(parts, wu0, wu1, wu2, wu3, auxu, tail, wp)


def kernel(pos, vel, edge_index, params):
    f32 = jnp.float32
    h16 = jnp.concatenate(
        [pos, vel, jnp.zeros((N, HPAD - 4), f32)], axis=1)
    src = edge_index[0]
    dst = edge_index[1]

    auxt = jnp.stack([
        params['bm0'], params['gm1'], params['hm1'],
        params['bm1'], params['gm2'], params['hm2'],
        params['bm2'], params['gm3'], params['hm3'],
        jnp.concatenate([params['bm3'], jnp.zeros((EMB - LOUT,), f32)]),
    ], axis=1)

    wu0 = jnp.concatenate(
        [params['Wu0'], jnp.zeros((HPAD - LOUT, EMB), f32)], axis=0)
    auxu = jnp.stack([
        params['bu0'], params['gu1'], params['hu1'],
        params['bu1'], params['gu2'], params['hu2'],
        params['bu2'], params['gu3'], params['hu3'],
    ], axis=0)
    tail = jnp.stack([
        params['bu3'], params['gu4'], params['hu4'],
        jnp.concatenate([params['bp'], jnp.zeros((LOUT - ODIM,), f32)]),
    ], axis=0)

    gdst, gsrc = _sc_gather(h16, src, dst)
    payload_p = _tc_edge_mlp(gdst.reshape(E // 8, 128),
                             gsrc.reshape(E // 8, 128),
                             params['Wm0'], params['Wm1'],
                             params['Wm2'], params['Wm3'], auxt)
    zrows = jnp.zeros((RPS, HPAD), f32)
    parts = _sc_scatter(payload_p.reshape(E, HPAD), dst, zrows)
    out = _tc_node_mlp(parts, wu0, params['Wu1'], params['Wu2'],
                       params['Wu3'], auxu, tail, params['Wp'])
    return out


# speedup vs baseline: 1.1166x; 1.0738x over previous
import functools

import jax
import jax.numpy as jnp
from jax import lax
from jax.experimental import pallas as pl
from jax.experimental.pallas import tpu as pltpu
from jax.experimental.pallas import tpu_sc as plsc

N = 10000
E = 160000
EMB = 64
LOUT = 4
ODIM = 2
HPAD = 16

NC = 2
NS = 16
NW = NC * NS
CH = E // NW
CHG = 1000
RPS = N // NS

BLK = 6400
NBLK = E // BLK
PB = BLK // 8

def _sc_mesh():
    return plsc.VectorSubcoreMesh(core_axis_name="c", subcore_axis_name="s",
                                  num_cores=NC, num_subcores=NS)


@jax.jit
def _sc_gather(h16, src, dst):
    @functools.partial(
        pl.kernel,
        out_type=(jax.ShapeDtypeStruct((E, HPAD), jnp.float32),
                  jax.ShapeDtypeStruct((E, HPAD), jnp.float32)),
        mesh=_sc_mesh(),
        compiler_params=pltpu.CompilerParams(use_tc_tiling_on_sc=False),
        scratch_types=[
            pltpu.VMEM((CH // CHG, CHG), jnp.int32),
            pltpu.VMEM((CH // CHG, CHG), jnp.int32),
            pltpu.VMEM((CHG, HPAD), jnp.float32),
            pltpu.VMEM((CHG, HPAD), jnp.float32),
            pltpu.SemaphoreType.DMA,
            pltpu.SemaphoreType.DMA,
            pltpu.SemaphoreType.DMA,
        ],
    )
    def k(h_hbm, src_hbm, dst_hbm, gdst_hbm, gsrc_hbm,
          idx_d, idx_s, buf0, buf1, sem_i, sem_g, sem_w):
        wid = lax.axis_index("s") * NC + lax.axis_index("c")
        base = wid * CH
        cpi_d = pltpu.async_copy(dst_hbm.at[wid], idx_d, sem_i)
        cpi_s = pltpu.async_copy(src_hbm.at[wid], idx_s, sem_i)
        cpi_d.wait()
        cpi_s.wait()

        nch = CH // CHG
        bufs = (buf0, buf1)
        outs = []
        for j in range(2 * nch):
            idx = idx_d if j < nch else idx_s
            ohbm = gdst_hbm if j < nch else gsrc_hbm
            c = (j % nch) * CHG
            buf = bufs[j % 2]
            if j >= 2:
                outs[j - 2].wait()
            pltpu.async_copy(h_hbm.at[idx.at[j % nch]], buf, sem_g).wait()
            outs.append(
                pltpu.async_copy(buf, ohbm.at[pl.ds(base + c, CHG)], sem_w))
        outs[-2].wait()
        outs[-1].wait()

    return k(h16, src.reshape(NW, CH // CHG, CHG),
             dst.reshape(NW, CH // CHG, CHG))


@jax.jit
def _sc_scatter(payload, dst, zrows):
    @functools.partial(
        pl.kernel,
        out_type=jax.ShapeDtypeStruct((NC, N, HPAD), jnp.float32),
        mesh=_sc_mesh(),
        compiler_params=pltpu.CompilerParams(use_tc_tiling_on_sc=False),
        scratch_types=[
            pltpu.VMEM((CH // CHG, CHG), jnp.int32),
            pltpu.VMEM((CHG, HPAD), jnp.float32),
            pltpu.VMEM((CHG, HPAD), jnp.float32),
            pltpu.VMEM_SHARED((N, HPAD), jnp.float32),
            pltpu.SemaphoreType.DMA,
            pltpu.SemaphoreType.DMA,
        ],
    )
    def k(pay_hbm, dst_hbm, z_hbm, out_hbm, idx_v, buf0, buf1, acc,
          sem_i, sem_p):
        c = lax.axis_index("c")
        s = lax.axis_index("s")
        wid = s * NC + c
        base = wid * CH
        cpz = pltpu.async_copy(z_hbm, acc.at[pl.ds(s * RPS, RPS)], sem_i)
        cpi = pltpu.async_copy(dst_hbm.at[wid], idx_v, sem_i)
        nch = CH // CHG
        bufs = (buf0, buf1)
        loads = [pltpu.async_copy(pay_hbm.at[pl.ds(base + j * CHG, CHG)],
                                  bufs[j % 2], sem_p) for j in range(2)]
        cpz.wait()
        cpi.wait()
        plsc.subcore_barrier()
        for j in range(nch):
            loads[j].wait()
            pltpu.sync_copy(bufs[j % 2], acc.at[idx_v.at[j]], add=True)
            if j + 2 < nch:
                loads.append(
                    pltpu.async_copy(pay_hbm.at[pl.ds(base + (j + 2) * CHG,
                                                      CHG)],
                                     bufs[j % 2], sem_p))
        plsc.subcore_barrier()
        pltpu.sync_copy(acc.at[pl.ds(s * RPS, RPS)],
                        out_hbm.at[c].at[pl.ds(s * RPS, RPS)])

    return k(payload, dst.reshape(NW, CH // CHG, CHG), zrows)


def _unpack(xp):
    t = jnp.transpose(xp)
    return jnp.concatenate(
        [t[HPAD * g:HPAD * (g + 1), :] for g in range(8)], axis=1)


def _pack(x):
    t = jnp.concatenate(
        [x[:, PB * g:PB * (g + 1)] for g in range(8)], axis=0)
    return jnp.transpose(t)


def _mm(w, x):
    return lax.dot_general(w, x, (((0,), (0,)), ((), ())),
                           preferred_element_type=jnp.float32)


def _edge_mlp_body(gdst, gsrc, w0, w1, w2, w3, auxt, out_ref,
                   act, stats):
    p = pl.program_id(0)
    i = pl.program_id(1)
    blk = pl.ds(i * BLK, BLK)
    inv_e = 1.0 / E

    @pl.when((p == 0) & (i == 0))
    def _():
        stats[...] = jnp.zeros_like(stats)

    def bn_tanh(x, li, g_col, h_col):
        m = stats[:, 2 * li:2 * li + 1] * inv_e
        v = stats[:, 2 * li + 1:2 * li + 2] * inv_e - m * m
        a = auxt[:, g_col:g_col + 1] * jax.lax.rsqrt(v + 1e-5)
        cc = auxt[:, h_col:h_col + 1] - m * a
        return jnp.tanh(x * a + cc)

    def put_stats(li, x):
        stats[:, 2 * li:2 * li + 1] += jnp.sum(x, axis=1, keepdims=True)
        stats[:, 2 * li + 1:2 * li + 2] += jnp.sum(x * x, axis=1,
                                                   keepdims=True)

    @pl.when(p == 0)
    def _():
        hdt = _unpack(gdst[...] - gsrc[...])
        x1 = _mm(w0[...], hdt[0:4, :]) + auxt[:, 0:1]
        act[:, blk] = x1
        put_stats(0, x1)

    @pl.when(p == 1)
    def _():
        t1 = bn_tanh(act[:, blk], 0, 1, 2)
        x2 = _mm(w1[...], t1) + auxt[:, 3:4]
        act[:, blk] = x2
        put_stats(1, x2)

    @pl.when(p == 2)
    def _():
        t2 = bn_tanh(act[:, blk], 1, 4, 5)
        x3 = _mm(w2[...], t2) + auxt[:, 6:7]
        act[:, blk] = x3
        put_stats(2, x3)

    @pl.when(p == 3)
    def _():
        t3 = bn_tanh(act[:, blk], 2, 7, 8)
        pay4 = _mm(w3[...], t3) + auxt[0:4, 9:10]
        hdt = _unpack(gdst[...] - gsrc[...])
        mask = jnp.all(hdt == 0.0, axis=0, keepdims=True)
        pay4 = jnp.where(mask, 0.0, pay4)
        pay = jnp.concatenate(
            [pay4, jnp.ones((1, BLK), jnp.float32),
             jnp.zeros((HPAD - LOUT - 1, BLK), jnp.float32)], axis=0)
        out_ref[...] = _pack(pay)


@jax.jit
def _tc_edge_mlp(gdst_p, gsrc_p, w0, w1, w2, w3, auxt):
    edge_map = lambda p, i: (jnp.where((p == 0) | (p == 3), i, 0), 0)
    rep = lambda p, i: (0, 0)
    return pl.pallas_call(
        _edge_mlp_body,
        grid=(4, NBLK),
        in_specs=[
            pl.BlockSpec((PB, 128), edge_map),
            pl.BlockSpec((PB, 128), edge_map),
            pl.BlockSpec((4, EMB), rep),
            pl.BlockSpec((EMB, EMB), rep),
            pl.BlockSpec((EMB, EMB), rep),
            pl.BlockSpec((EMB, LOUT), rep),
            pl.BlockSpec((EMB, 10), rep),
        ],
        out_specs=pl.BlockSpec((PB, 128),
                               lambda p, i: (jnp.where(p == 3, i, 0), 0)),
        out_shape=jax.ShapeDtypeStruct((E // 8, 128), jnp.float32),
        scratch_shapes=[
            pltpu.VMEM((EMB, E), jnp.float32),
            pltpu.VMEM((EMB, 8), jnp.float32),
        ],
    )(gdst_p, gsrc_p, w0, w1, w2, w3, auxt)


NPB = N * HPAD // 128


def _node_mlp_body(parts, wu0, wu1, wu2, wu3, auxu, tailt, wp, out_ref):
    def unpack_n(xp):
        t = jnp.transpose(xp)
        return jnp.concatenate(
            [t[HPAD * g:HPAD * (g + 1), :] for g in range(8)], axis=1)

    s = unpack_n(parts[0]) + unpack_n(parts[1])
    cnt = jnp.maximum(s[LOUT:LOUT + 1, :], 1.0)
    aggr = jnp.concatenate([s[0:2, :], s[2:4, :] / cnt], axis=0)

    def bn_tanh(x, g, h):
        m = jnp.mean(x, axis=1, keepdims=True)
        v = jnp.mean((x - m) * (x - m), axis=1, keepdims=True)
        a = g * jax.lax.rsqrt(v + 1e-5)
        return jnp.tanh(x * a + (h - m * a))

    x = _mm(wu0[...], aggr) + auxu[:, 0:1]
    x = bn_tanh(x, auxu[:, 1:2], auxu[:, 2:3])
    x = _mm(wu1[...], x) + auxu[:, 3:4]
    x = bn_tanh(x, auxu[:, 4:5], auxu[:, 5:6])
    x = _mm(wu2[...], x) + auxu[:, 6:7]
    x = bn_tanh(x, auxu[:, 7:8], auxu[:, 8:9])
    x = _mm(wu3[...], x) + tailt[:, 0:1]
    x = bn_tanh(x, tailt[:, 1:2], tailt[:, 2:3])
    o = _mm(wp[...], x) + tailt[0:ODIM, 3:4]
    o16 = jnp.concatenate(
        [o, jnp.zeros((HPAD - ODIM, N), jnp.float32)], axis=0)
    t = jnp.concatenate(
        [o16[:, (N // 8) * g:(N // 8) * (g + 1)] for g in range(8)], axis=0)
    out_ref[...] = jnp.transpose(t)


@jax.jit
def _tc_node_mlp(parts, wu0, wu1, wu2, wu3, auxu, tailt, wp):
    return pl.pallas_call(
        _node_mlp_body,
        out_shape=jax.ShapeDtypeStruct((NPB, 128), jnp.float32),
    )(parts, wu0, wu1, wu2, wu3, auxu, tailt, wp)


def kernel(pos, vel, edge_index, params):
    f32 = jnp.float32
    h16 = jnp.concatenate(
        [pos, vel, jnp.zeros((N, HPAD - 4), f32)], axis=1)
    src = edge_index[0]
    dst = edge_index[1]

    auxt = jnp.stack([
        params['bm0'], params['gm1'], params['hm1'],
        params['bm1'], params['gm2'], params['hm2'],
        params['bm2'], params['gm3'], params['hm3'],
        jnp.concatenate([params['bm3'], jnp.zeros((EMB - LOUT,), f32)]),
    ], axis=1)

    auxu = jnp.stack([
        params['bu0'], params['gu1'], params['hu1'],
        params['bu1'], params['gu2'], params['hu2'],
        params['bu2'], params['gu3'], params['hu3'],
    ], axis=1)
    tailt = jnp.stack([
        params['bu3'], params['gu4'], params['hu4'],
        jnp.concatenate([params['bp'], jnp.zeros((LOUT - ODIM,), f32)]),
    ], axis=1)

    gdst, gsrc = _sc_gather(h16, src, dst)
    payload_p = _tc_edge_mlp(gdst.reshape(E // 8, 128),
                             gsrc.reshape(E // 8, 128),
                             params['Wm0'], params['Wm1'],
                             params['Wm2'], params['Wm3'], auxt)
    zrows = jnp.zeros((RPS, HPAD), f32)
    parts = _sc_scatter(payload_p.reshape(E, HPAD), dst, zrows)
    out = _tc_node_mlp(parts.reshape(NC, NPB, 128), params['Wu0'],
                       params['Wu1'], params['Wu2'], params['Wu3'],
                       auxu, tailt, params['Wp'])
    return out.reshape(N, HPAD)[:, :ODIM]


# speedup vs baseline: 1.1931x; 1.0684x over previous
import functools

import jax
import jax.numpy as jnp
from jax import lax
from jax.experimental import pallas as pl
from jax.experimental.pallas import tpu as pltpu
from jax.experimental.pallas import tpu_sc as plsc

N = 10000
E = 160000
EMB = 64
LOUT = 4
ODIM = 2
HPAD = 8

NC = 2
NS = 16
NW = NC * NS
CH = E // NW
CHG = 1000
RPS = N // NS

BLK = 6400
NBLK = E // BLK
G = 128 // HPAD
PB = BLK * HPAD // 128
EPB = E * HPAD // 128

def _sc_mesh():
    return plsc.VectorSubcoreMesh(core_axis_name="c", subcore_axis_name="s",
                                  num_cores=NC, num_subcores=NS)


@jax.jit
def _sc_gather(h16, src, dst):
    @functools.partial(
        pl.kernel,
        out_type=(jax.ShapeDtypeStruct((E, HPAD), jnp.float32),
                  jax.ShapeDtypeStruct((E, HPAD), jnp.float32)),
        mesh=_sc_mesh(),
        compiler_params=pltpu.CompilerParams(use_tc_tiling_on_sc=False),
        scratch_types=[
            pltpu.VMEM((CH // CHG, CHG), jnp.int32),
            pltpu.VMEM((CH // CHG, CHG), jnp.int32),
            pltpu.VMEM((CHG, HPAD), jnp.float32),
            pltpu.VMEM((CHG, HPAD), jnp.float32),
            pltpu.SemaphoreType.DMA,
            pltpu.SemaphoreType.DMA,
            pltpu.SemaphoreType.DMA,
        ],
    )
    def k(h_hbm, src_hbm, dst_hbm, gdst_hbm, gsrc_hbm,
          idx_d, idx_s, buf0, buf1, sem_i, sem_g, sem_w):
        wid = lax.axis_index("s") * NC + lax.axis_index("c")
        base = wid * CH
        cpi_d = pltpu.async_copy(dst_hbm.at[wid], idx_d, sem_i)
        cpi_s = pltpu.async_copy(src_hbm.at[wid], idx_s, sem_i)
        cpi_d.wait()
        cpi_s.wait()

        nch = CH // CHG
        bufs = (buf0, buf1)
        outs = []
        for j in range(2 * nch):
            idx = idx_d if j < nch else idx_s
            ohbm = gdst_hbm if j < nch else gsrc_hbm
            c = (j % nch) * CHG
            buf = bufs[j % 2]
            if j >= 2:
                outs[j - 2].wait()
            pltpu.async_copy(h_hbm.at[idx.at[j % nch]], buf, sem_g).wait()
            outs.append(
                pltpu.async_copy(buf, ohbm.at[pl.ds(base + c, CHG)], sem_w))
        outs[-2].wait()
        outs[-1].wait()

    return k(h16, src.reshape(NW, CH // CHG, CHG),
             dst.reshape(NW, CH // CHG, CHG))


@jax.jit
def _sc_scatter(payload, dst, zrows):
    @functools.partial(
        pl.kernel,
        out_type=jax.ShapeDtypeStruct((NC, N, HPAD), jnp.float32),
        mesh=_sc_mesh(),
        compiler_params=pltpu.CompilerParams(use_tc_tiling_on_sc=False),
        scratch_types=[
            pltpu.VMEM((CH // CHG, CHG), jnp.int32),
            pltpu.VMEM((CHG, HPAD), jnp.float32),
            pltpu.VMEM((CHG, HPAD), jnp.float32),
            pltpu.VMEM_SHARED((N, HPAD), jnp.float32),
            pltpu.SemaphoreType.DMA,
            pltpu.SemaphoreType.DMA,
        ],
    )
    def k(pay_hbm, dst_hbm, z_hbm, out_hbm, idx_v, buf0, buf1, acc,
          sem_i, sem_p):
        c = lax.axis_index("c")
        s = lax.axis_index("s")
        wid = s * NC + c
        base = wid * CH
        cpz = pltpu.async_copy(z_hbm, acc.at[pl.ds(s * RPS, RPS)], sem_i)
        cpi = pltpu.async_copy(dst_hbm.at[wid], idx_v, sem_i)
        nch = CH // CHG
        bufs = (buf0, buf1)
        loads = [pltpu.async_copy(pay_hbm.at[pl.ds(base + j * CHG, CHG)],
                                  bufs[j % 2], sem_p) for j in range(2)]
        cpz.wait()
        cpi.wait()
        plsc.subcore_barrier()
        for j in range(nch):
            loads[j].wait()
            pltpu.sync_copy(bufs[j % 2], acc.at[idx_v.at[j]], add=True)
            if j + 2 < nch:
                loads.append(
                    pltpu.async_copy(pay_hbm.at[pl.ds(base + (j + 2) * CHG,
                                                      CHG)],
                                     bufs[j % 2], sem_p))
        plsc.subcore_barrier()
        pltpu.sync_copy(acc.at[pl.ds(s * RPS, RPS)],
                        out_hbm.at[c].at[pl.ds(s * RPS, RPS)])

    return k(payload, dst.reshape(NW, CH // CHG, CHG), zrows)


def _unpack(xp):
    t = jnp.transpose(xp)
    return jnp.concatenate(
        [t[HPAD * g:HPAD * (g + 1), :] for g in range(G)], axis=1)


def _pack(x):
    t = jnp.concatenate(
        [x[:, PB * g:PB * (g + 1)] for g in range(G)], axis=0)
    return jnp.transpose(t)


def _mm(w, x):
    return lax.dot_general(w, x, (((0,), (0,)), ((), ())),
                           preferred_element_type=jnp.float32)


def _edge_mlp_body(gdst, gsrc, w0, w1, w2, w3, auxt, out_ref,
                   act, stats):
    p = pl.program_id(0)
    i = pl.program_id(1)
    blk = pl.ds(i * BLK, BLK)
    inv_e = 1.0 / E

    @pl.when((p == 0) & (i == 0))
    def _():
        stats[...] = jnp.zeros_like(stats)

    def bn_tanh(x, li, g_col, h_col):
        m = stats[:, 2 * li:2 * li + 1] * inv_e
        v = stats[:, 2 * li + 1:2 * li + 2] * inv_e - m * m
        a = auxt[:, g_col:g_col + 1] * jax.lax.rsqrt(v + 1e-5)
        cc = auxt[:, h_col:h_col + 1] - m * a
        return jnp.tanh(x * a + cc)

    def put_stats(li, x):
        stats[:, 2 * li:2 * li + 1] += jnp.sum(x, axis=1, keepdims=True)
        stats[:, 2 * li + 1:2 * li + 2] += jnp.sum(x * x, axis=1,
                                                   keepdims=True)

    @pl.when(p == 0)
    def _():
        hdt = _unpack(gdst[...] - gsrc[...])
        x1 = _mm(w0[...], hdt[0:4, :]) + auxt[:, 0:1]
        act[:, blk] = x1
        put_stats(0, x1)

    @pl.when(p == 1)
    def _():
        t1 = bn_tanh(act[:, blk], 0, 1, 2)
        x2 = _mm(w1[...], t1) + auxt[:, 3:4]
        act[:, blk] = x2
        put_stats(1, x2)

    @pl.when(p == 2)
    def _():
        t2 = bn_tanh(act[:, blk], 1, 4, 5)
        x3 = _mm(w2[...], t2) + auxt[:, 6:7]
        act[:, blk] = x3
        put_stats(2, x3)

    @pl.when(p == 3)
    def _():
        t3 = bn_tanh(act[:, blk], 2, 7, 8)
        pay4 = _mm(w3[...], t3) + auxt[0:4, 9:10]
        hdt = _unpack(gdst[...] - gsrc[...])
        mask = jnp.all(hdt == 0.0, axis=0, keepdims=True)
        pay4 = jnp.where(mask, 0.0, pay4)
        pay = jnp.concatenate(
            [pay4, jnp.ones((1, BLK), jnp.float32),
             jnp.zeros((HPAD - LOUT - 1, BLK), jnp.float32)], axis=0)
        out_ref[...] = _pack(pay)


@jax.jit
def _tc_edge_mlp(gdst_p, gsrc_p, w0, w1, w2, w3, auxt):
    edge_map = lambda p, i: (jnp.where((p == 0) | (p == 3), i, 0), 0)
    rep = lambda p, i: (0, 0)
    return pl.pallas_call(
        _edge_mlp_body,
        grid=(4, NBLK),
        in_specs=[
            pl.BlockSpec((PB, 128), edge_map),
            pl.BlockSpec((PB, 128), edge_map),
            pl.BlockSpec((4, EMB), rep),
            pl.BlockSpec((EMB, EMB), rep),
            pl.BlockSpec((EMB, EMB), rep),
            pl.BlockSpec((EMB, LOUT), rep),
            pl.BlockSpec((EMB, 10), rep),
        ],
        out_specs=pl.BlockSpec((PB, 128),
                               lambda p, i: (jnp.where(p == 3, i, 0), 0)),
        out_shape=jax.ShapeDtypeStruct((EPB, 128), jnp.float32),
        scratch_shapes=[
            pltpu.VMEM((EMB, E), jnp.float32),
            pltpu.VMEM((EMB, 8), jnp.float32),
        ],
    )(gdst_p, gsrc_p, w0, w1, w2, w3, auxt)


NPB = N * HPAD // 128


def _node_mlp_body(parts, wu0, wu1, wu2, wu3, auxu, tailt, wp, out_ref):
    def unpack_n(xp):
        t = jnp.transpose(xp)
        return jnp.concatenate(
            [t[HPAD * g:HPAD * (g + 1), :] for g in range(G)], axis=1)

    s = unpack_n(parts[0]) + unpack_n(parts[1])
    cnt = jnp.maximum(s[LOUT:LOUT + 1, :], 1.0)
    aggr = jnp.concatenate([s[0:2, :], s[2:4, :] / cnt], axis=0)

    def bn_tanh(x, g, h):
        m = jnp.mean(x, axis=1, keepdims=True)
        v = jnp.mean((x - m) * (x - m), axis=1, keepdims=True)
        a = g * jax.lax.rsqrt(v + 1e-5)
        return jnp.tanh(x * a + (h - m * a))

    x = _mm(wu0[...], aggr) + auxu[:, 0:1]
    x = bn_tanh(x, auxu[:, 1:2], auxu[:, 2:3])
    x = _mm(wu1[...], x) + auxu[:, 3:4]
    x = bn_tanh(x, auxu[:, 4:5], auxu[:, 5:6])
    x = _mm(wu2[...], x) + auxu[:, 6:7]
    x = bn_tanh(x, auxu[:, 7:8], auxu[:, 8:9])
    x = _mm(wu3[...], x) + tailt[:, 0:1]
    x = bn_tanh(x, tailt[:, 1:2], tailt[:, 2:3])
    o = _mm(wp[...], x) + tailt[0:ODIM, 3:4]
    o16 = jnp.concatenate(
        [o, jnp.zeros((HPAD - ODIM, N), jnp.float32)], axis=0)
    t = jnp.concatenate(
        [o16[:, NPB * g:NPB * (g + 1)] for g in range(G)], axis=0)
    out_ref[...] = jnp.transpose(t)


@jax.jit
def _tc_node_mlp(parts, wu0, wu1, wu2, wu3, auxu, tailt, wp):
    return pl.pallas_call(
        _node_mlp_body,
        out_shape=jax.ShapeDtypeStruct((NPB, 128), jnp.float32),
    )(parts, wu0, wu1, wu2, wu3, auxu, tailt, wp)


def kernel(pos, vel, edge_index, params):
    f32 = jnp.float32
    h16 = jnp.concatenate(
        [pos, vel, jnp.zeros((N, HPAD - 4), f32)], axis=1)
    src = edge_index[0]
    dst = edge_index[1]

    auxt = jnp.stack([
        params['bm0'], params['gm1'], params['hm1'],
        params['bm1'], params['gm2'], params['hm2'],
        params['bm2'], params['gm3'], params['hm3'],
        jnp.concatenate([params['bm3'], jnp.zeros((EMB - LOUT,), f32)]),
    ], axis=1)

    auxu = jnp.stack([
        params['bu0'], params['gu1'], params['hu1'],
        params['bu1'], params['gu2'], params['hu2'],
        params['bu2'], params['gu3'], params['hu3'],
    ], axis=1)
    tailt = jnp.stack([
        params['bu3'], params['gu4'], params['hu4'],
        jnp.concatenate([params['bp'], jnp.zeros((LOUT - ODIM,), f32)]),
    ], axis=1)

    gdst, gsrc = _sc_gather(h16, src, dst)
    payload_p = _tc_edge_mlp(gdst.reshape(EPB, 128),
                             gsrc.reshape(EPB, 128),
                             params['Wm0'], params['Wm1'],
                             params['Wm2'], params['Wm3'], auxt)
    zrows = jnp.zeros((RPS, HPAD), f32)
    parts = _sc_scatter(payload_p.reshape(E, HPAD), dst, zrows)
    out = _tc_node_mlp(parts.reshape(NC, NPB, 128), params['Wu0'],
                       params['Wu1'], params['Wu2'], params['Wu3'],
                       auxu, tailt, params['Wp'])
    return out.reshape(N, HPAD)[:, :ODIM]


# speedup vs baseline: 1.1937x; 1.0006x over previous
import functools

import jax
import jax.numpy as jnp
from jax import lax
from jax.experimental import pallas as pl
from jax.experimental.pallas import tpu as pltpu
from jax.experimental.pallas import tpu_sc as plsc

N = 10000
E = 160000
EMB = 64
LOUT = 4
ODIM = 2
HPAD = 8

NC = 2
NS = 16
NW = NC * NS
CH = E // NW
CHG = 1000
RPS = N // NS

BLK = 6400
NBLK = E // BLK
G = 128 // HPAD
PB = BLK * HPAD // 128
EPB = E * HPAD // 128

def _sc_mesh():
    return plsc.VectorSubcoreMesh(core_axis_name="c", subcore_axis_name="s",
                                  num_cores=NC, num_subcores=NS)


@jax.jit
def _sc_gather(h16, src, dst):
    @functools.partial(
        pl.kernel,
        out_type=(jax.ShapeDtypeStruct((E, HPAD), jnp.float32),
                  jax.ShapeDtypeStruct((E, HPAD), jnp.float32)),
        mesh=_sc_mesh(),
        compiler_params=pltpu.CompilerParams(use_tc_tiling_on_sc=False),
        scratch_types=[
            pltpu.VMEM((CH // CHG, CHG), jnp.int32),
            pltpu.VMEM((CH // CHG, CHG), jnp.int32),
            pltpu.VMEM((CHG, HPAD), jnp.float32),
            pltpu.VMEM((CHG, HPAD), jnp.float32),
            pltpu.SemaphoreType.DMA,
            pltpu.SemaphoreType.DMA,
            pltpu.SemaphoreType.DMA,
        ],
    )
    def k(h_hbm, src_hbm, dst_hbm, gdst_hbm, gsrc_hbm,
          idx_d, idx_s, buf0, buf1, sem_i, sem_g, sem_w):
        wid = lax.axis_index("s") * NC + lax.axis_index("c")
        base = wid * CH
        cpi_d = pltpu.async_copy(dst_hbm.at[wid], idx_d, sem_i)
        cpi_s = pltpu.async_copy(src_hbm.at[wid], idx_s, sem_i)
        cpi_d.wait()
        cpi_s.wait()

        nch = CH // CHG
        bufs = (buf0, buf1)
        outs = []
        for j in range(2 * nch):
            idx = idx_d if j < nch else idx_s
            ohbm = gdst_hbm if j < nch else gsrc_hbm
            c = (j % nch) * CHG
            buf = bufs[j % 2]
            if j >= 2:
                outs[j - 2].wait()
            pltpu.async_copy(h_hbm.at[idx.at[j % nch]], buf, sem_g).wait()
            outs.append(
                pltpu.async_copy(buf, ohbm.at[pl.ds(base + c, CHG)], sem_w))
        outs[-2].wait()
        outs[-1].wait()

    return k(h16, src.reshape(NW, CH // CHG, CHG),
             dst.reshape(NW, CH // CHG, CHG))


@jax.jit
def _sc_scatter(payload, dst, zrows):
    @functools.partial(
        pl.kernel,
        out_type=jax.ShapeDtypeStruct((NC, N, HPAD), jnp.float32),
        mesh=_sc_mesh(),
        compiler_params=pltpu.CompilerParams(use_tc_tiling_on_sc=False),
        scratch_types=[
            pltpu.VMEM((CH // CHG, CHG), jnp.int32),
            pltpu.VMEM((CHG, HPAD), jnp.float32),
            pltpu.VMEM((CHG, HPAD), jnp.float32),
            pltpu.VMEM_SHARED((N, HPAD), jnp.float32),
            pltpu.SemaphoreType.DMA,
            pltpu.SemaphoreType.DMA,
        ],
    )
    def k(pay_hbm, dst_hbm, z_hbm, out_hbm, idx_v, buf0, buf1, acc,
          sem_i, sem_p):
        c = lax.axis_index("c")
        s = lax.axis_index("s")
        wid = s * NC + c
        base = wid * CH
        cpz = pltpu.async_copy(z_hbm, acc.at[pl.ds(s * RPS, RPS)], sem_i)
        cpi = pltpu.async_copy(dst_hbm.at[wid], idx_v, sem_i)
        nch = CH // CHG
        bufs = (buf0, buf1)
        loads = [pltpu.async_copy(pay_hbm.at[pl.ds(base + j * CHG, CHG)],
                                  bufs[j % 2], sem_p) for j in range(2)]
        cpz.wait()
        cpi.wait()
        plsc.subcore_barrier()
        for j in range(nch):
            loads[j].wait()
            pltpu.sync_copy(bufs[j % 2], acc.at[idx_v.at[j]], add=True)
            if j + 2 < nch:
                loads.append(
                    pltpu.async_copy(pay_hbm.at[pl.ds(base + (j + 2) * CHG,
                                                      CHG)],
                                     bufs[j % 2], sem_p))
        plsc.subcore_barrier()
        pltpu.sync_copy(acc.at[pl.ds(s * RPS, RPS)],
                        out_hbm.at[c].at[pl.ds(s * RPS, RPS)])

    return k(payload, dst.reshape(NW, CH // CHG, CHG), zrows)


def _unpack(xp):
    t = jnp.transpose(xp)
    return jnp.concatenate(
        [t[HPAD * g:HPAD * (g + 1), :] for g in range(G)], axis=1)


def _pack(x):
    t = jnp.concatenate(
        [x[:, PB * g:PB * (g + 1)] for g in range(G)], axis=0)
    return jnp.transpose(t)


def _mm(w, x):
    return lax.dot_general(w, x, (((0,), (0,)), ((), ())),
                           preferred_element_type=jnp.float32)


def _edge_mlp_body(gdst, gsrc, w0, w1, w2, w3, auxt, out_ref,
                   act, stats):
    p = pl.program_id(0)
    i = pl.program_id(1)
    blk = pl.ds(i * BLK, BLK)
    inv_e = 1.0 / E

    @pl.when((p == 0) & (i == 0))
    def _():
        stats[...] = jnp.zeros_like(stats)

    def bn_tanh(x, li, g_col, h_col):
        m = stats[:, 2 * li:2 * li + 1] * inv_e
        v = stats[:, 2 * li + 1:2 * li + 2] * inv_e - m * m
        a = auxt[:, g_col:g_col + 1] * jax.lax.rsqrt(v + 1e-5)
        cc = auxt[:, h_col:h_col + 1] - m * a
        return jnp.tanh(x.astype(jnp.float32) * a + cc)

    def put_stats(li, x):
        stats[:, 2 * li:2 * li + 1] += jnp.sum(x, axis=1, keepdims=True)
        stats[:, 2 * li + 1:2 * li + 2] += jnp.sum(x * x, axis=1,
                                                   keepdims=True)

    @pl.when(p == 0)
    def _():
        hdt = _unpack(gdst[...] - gsrc[...])
        x1 = _mm(w0[...], hdt[0:4, :]) + auxt[:, 0:1]
        act[:, blk] = x1.astype(jnp.bfloat16)
        put_stats(0, x1)

    @pl.when(p == 1)
    def _():
        t1 = bn_tanh(act[:, blk], 0, 1, 2)
        x2 = _mm(w1[...], t1) + auxt[:, 3:4]
        act[:, blk] = x2.astype(jnp.bfloat16)
        put_stats(1, x2)

    @pl.when(p == 2)
    def _():
        t2 = bn_tanh(act[:, blk], 1, 4, 5)
        x3 = _mm(w2[...], t2) + auxt[:, 6:7]
        act[:, blk] = x3.astype(jnp.bfloat16)
        put_stats(2, x3)

    @pl.when(p == 3)
    def _():
        t3 = bn_tanh(act[:, blk], 2, 7, 8)
        pay4 = _mm(w3[...], t3) + auxt[0:4, 9:10]
        hdt = _unpack(gdst[...] - gsrc[...])
        mask = jnp.all(hdt == 0.0, axis=0, keepdims=True)
        pay4 = jnp.where(mask, 0.0, pay4)
        pay = jnp.concatenate(
            [pay4, jnp.ones((1, BLK), jnp.float32),
             jnp.zeros((HPAD - LOUT - 1, BLK), jnp.float32)], axis=0)
        out_ref[...] = _pack(pay)


@jax.jit
def _tc_edge_mlp(gdst_p, gsrc_p, w0, w1, w2, w3, auxt):
    edge_map = lambda p, i: (jnp.where((p == 0) | (p == 3), i, 0), 0)
    rep = lambda p, i: (0, 0)
    return pl.pallas_call(
        _edge_mlp_body,
        grid=(4, NBLK),
        in_specs=[
            pl.BlockSpec((PB, 128), edge_map),
            pl.BlockSpec((PB, 128), edge_map),
            pl.BlockSpec((4, EMB), rep),
            pl.BlockSpec((EMB, EMB), rep),
            pl.BlockSpec((EMB, EMB), rep),
            pl.BlockSpec((EMB, LOUT), rep),
            pl.BlockSpec((EMB, 10), rep),
        ],
        out_specs=pl.BlockSpec((PB, 128),
                               lambda p, i: (jnp.where(p == 3, i, 0), 0)),
        out_shape=jax.ShapeDtypeStruct((EPB, 128), jnp.float32),
        scratch_shapes=[
            pltpu.VMEM((EMB, E), jnp.bfloat16),
            pltpu.VMEM((EMB, 8), jnp.float32),
        ],
    )(gdst_p, gsrc_p, w0, w1, w2, w3, auxt)


NPB = N * HPAD // 128


def _node_mlp_body(parts, wu0, wu1, wu2, wu3, auxu, tailt, wp, out_ref):
    def unpack_n(xp):
        t = jnp.transpose(xp)
        return jnp.concatenate(
            [t[HPAD * g:HPAD * (g + 1), :] for g in range(G)], axis=1)

    s = unpack_n(parts[0]) + unpack_n(parts[1])
    cnt = jnp.maximum(s[LOUT:LOUT + 1, :], 1.0)
    aggr = jnp.concatenate([s[0:2, :], s[2:4, :] / cnt], axis=0)

    def bn_tanh(x, g, h):
        m = jnp.mean(x, axis=1, keepdims=True)
        v = jnp.mean((x - m) * (x - m), axis=1, keepdims=True)
        a = g * jax.lax.rsqrt(v + 1e-5)
        return jnp.tanh(x * a + (h - m * a))

    x = _mm(wu0[...], aggr) + auxu[:, 0:1]
    x = bn_tanh(x, auxu[:, 1:2], auxu[:, 2:3])
    x = _mm(wu1[...], x) + auxu[:, 3:4]
    x = bn_tanh(x, auxu[:, 4:5], auxu[:, 5:6])
    x = _mm(wu2[...], x) + auxu[:, 6:7]
    x = bn_tanh(x, auxu[:, 7:8], auxu[:, 8:9])
    x = _mm(wu3[...], x) + tailt[:, 0:1]
    x = bn_tanh(x, tailt[:, 1:2], tailt[:, 2:3])
    o = _mm(wp[...], x) + tailt[0:ODIM, 3:4]
    o16 = jnp.concatenate(
        [o, jnp.zeros((HPAD - ODIM, N), jnp.float32)], axis=0)
    t = jnp.concatenate(
        [o16[:, NPB * g:NPB * (g + 1)] for g in range(G)], axis=0)
    out_ref[...] = jnp.transpose(t)


@jax.jit
def _tc_node_mlp(parts, wu0, wu1, wu2, wu3, auxu, tailt, wp):
    return pl.pallas_call(
        _node_mlp_body,
        out_shape=jax.ShapeDtypeStruct((NPB, 128), jnp.float32),
    )(parts, wu0, wu1, wu2, wu3, auxu, tailt, wp)


def kernel(pos, vel, edge_index, params):
    f32 = jnp.float32
    h16 = jnp.concatenate(
        [pos, vel, jnp.zeros((N, HPAD - 4), f32)], axis=1)
    src = edge_index[0]
    dst = edge_index[1]

    auxt = jnp.stack([
        params['bm0'], params['gm1'], params['hm1'],
        params['bm1'], params['gm2'], params['hm2'],
        params['bm2'], params['gm3'], params['hm3'],
        jnp.concatenate([params['bm3'], jnp.zeros((EMB - LOUT,), f32)]),
    ], axis=1)

    auxu = jnp.stack([
        params['bu0'], params['gu1'], params['hu1'],
        params['bu1'], params['gu2'], params['hu2'],
        params['bu2'], params['gu3'], params['hu3'],
    ], axis=1)
    tailt = jnp.stack([
        params['bu3'], params['gu4'], params['hu4'],
        jnp.concatenate([params['bp'], jnp.zeros((LOUT - ODIM,), f32)]),
    ], axis=1)

    gdst, gsrc = _sc_gather(h16, src, dst)
    payload_p = _tc_edge_mlp(gdst.reshape(EPB, 128),
                             gsrc.reshape(EPB, 128),
                             params['Wm0'], params['Wm1'],
                             params['Wm2'], params['Wm3'], auxt)
    zrows = jnp.zeros((RPS, HPAD), f32)
    parts = _sc_scatter(payload_p.reshape(E, HPAD), dst, zrows)
    out = _tc_node_mlp(parts.reshape(NC, NPB, 128), params['Wu0'],
                       params['Wu1'], params['Wu2'], params['Wu3'],
                       auxu, tailt, params['Wp'])
    return out.reshape(N, HPAD)[:, :ODIM]


# speedup vs baseline: 1.4473x; 1.2124x over previous
import functools

import jax
import jax.numpy as jnp
from jax import lax
from jax.experimental import pallas as pl
from jax.experimental.pallas import tpu as pltpu
from jax.experimental.pallas import tpu_sc as plsc

N = 10000
E = 160000
EMB = 64
LOUT = 4
ODIM = 2
HPAD = 8

NC = 2
NS = 16
NW = NC * NS
CH = E // NW
CHG = 1000
RPS = N // NS

BLK = 16000
NBLK = E // BLK
G = 128 // HPAD
PB = BLK * HPAD // 128
EPB = E * HPAD // 128

def _sc_mesh():
    return plsc.VectorSubcoreMesh(core_axis_name="c", subcore_axis_name="s",
                                  num_cores=NC, num_subcores=NS)


@jax.jit
def _sc_gather(h16, src, dst):
    @functools.partial(
        pl.kernel,
        out_type=(jax.ShapeDtypeStruct((E, HPAD), jnp.float32),
                  jax.ShapeDtypeStruct((E, HPAD), jnp.float32)),
        mesh=_sc_mesh(),
        compiler_params=pltpu.CompilerParams(use_tc_tiling_on_sc=False),
        scratch_types=[
            pltpu.VMEM((CH // CHG, CHG), jnp.int32),
            pltpu.VMEM((CH // CHG, CHG), jnp.int32),
            pltpu.VMEM((CHG, HPAD), jnp.float32),
            pltpu.VMEM((CHG, HPAD), jnp.float32),
            pltpu.SemaphoreType.DMA,
            pltpu.SemaphoreType.DMA,
            pltpu.SemaphoreType.DMA,
        ],
    )
    def k(h_hbm, src_hbm, dst_hbm, gdst_hbm, gsrc_hbm,
          idx_d, idx_s, buf0, buf1, sem_i, sem_g, sem_w):
        wid = lax.axis_index("s") * NC + lax.axis_index("c")
        base = wid * CH
        cpi_d = pltpu.async_copy(dst_hbm.at[wid], idx_d, sem_i)
        cpi_s = pltpu.async_copy(src_hbm.at[wid], idx_s, sem_i)
        cpi_d.wait()
        cpi_s.wait()

        nch = CH // CHG
        bufs = (buf0, buf1)
        outs = []
        for j in range(2 * nch):
            idx = idx_d if j < nch else idx_s
            ohbm = gdst_hbm if j < nch else gsrc_hbm
            c = (j % nch) * CHG
            buf = bufs[j % 2]
            if j >= 2:
                outs[j - 2].wait()
            pltpu.async_copy(h_hbm.at[idx.at[j % nch]], buf, sem_g).wait()
            outs.append(
                pltpu.async_copy(buf, ohbm.at[pl.ds(base + c, CHG)], sem_w))
        outs[-2].wait()
        outs[-1].wait()

    return k(h16, src.reshape(NW, CH // CHG, CHG),
             dst.reshape(NW, CH // CHG, CHG))


@jax.jit
def _sc_scatter(payload, dst, zrows):
    @functools.partial(
        pl.kernel,
        out_type=jax.ShapeDtypeStruct((NC, N, HPAD), jnp.float32),
        mesh=_sc_mesh(),
        compiler_params=pltpu.CompilerParams(use_tc_tiling_on_sc=False),
        scratch_types=[
            pltpu.VMEM((CH // CHG, CHG), jnp.int32),
            pltpu.VMEM((CHG, HPAD), jnp.float32),
            pltpu.VMEM((CHG, HPAD), jnp.float32),
            pltpu.VMEM_SHARED((N, HPAD), jnp.float32),
            pltpu.SemaphoreType.DMA,
            pltpu.SemaphoreType.DMA,
        ],
    )
    def k(pay_hbm, dst_hbm, z_hbm, out_hbm, idx_v, buf0, buf1, acc,
          sem_i, sem_p):
        c = lax.axis_index("c")
        s = lax.axis_index("s")
        wid = s * NC + c
        base = wid * CH
        cpz = pltpu.async_copy(z_hbm, acc.at[pl.ds(s * RPS, RPS)], sem_i)
        cpi = pltpu.async_copy(dst_hbm.at[wid], idx_v, sem_i)
        nch = CH // CHG
        bufs = (buf0, buf1)
        loads = [pltpu.async_copy(pay_hbm.at[pl.ds(base + j * CHG, CHG)],
                                  bufs[j % 2], sem_p) for j in range(2)]
        cpz.wait()
        cpi.wait()
        plsc.subcore_barrier()
        for j in range(nch):
            loads[j].wait()
            pltpu.sync_copy(bufs[j % 2], acc.at[idx_v.at[j]], add=True)
            if j + 2 < nch:
                loads.append(
                    pltpu.async_copy(pay_hbm.at[pl.ds(base + (j + 2) * CHG,
                                                      CHG)],
                                     bufs[j % 2], sem_p))
        plsc.subcore_barrier()
        pltpu.sync_copy(acc.at[pl.ds(s * RPS, RPS)],
                        out_hbm.at[c].at[pl.ds(s * RPS, RPS)])

    return k(payload, dst.reshape(NW, CH // CHG, CHG), zrows)


def _unpack(xp):
    t = jnp.transpose(xp)
    return jnp.concatenate(
        [t[HPAD * g:HPAD * (g + 1), :] for g in range(G)], axis=1)


def _pack(x):
    t = jnp.concatenate(
        [x[:, PB * g:PB * (g + 1)] for g in range(G)], axis=0)
    return jnp.transpose(t)


def _mm(w, x):
    return lax.dot_general(w, x, (((0,), (0,)), ((), ())),
                           preferred_element_type=jnp.float32)


def _edge_mlp_body(gdst, gsrc, w0, w1, w2, w3, auxt, out_ref,
                   act, stats):
    p = pl.program_id(0)
    i = pl.program_id(1)
    blk = pl.ds(i * BLK, BLK)
    inv_e = 1.0 / E

    @pl.when((p == 0) & (i == 0))
    def _():
        stats[...] = jnp.zeros_like(stats)

    def bn_tanh(x, li, g_col, h_col):
        m = stats[:, 2 * li:2 * li + 1] * inv_e
        v = stats[:, 2 * li + 1:2 * li + 2] * inv_e - m * m
        a = auxt[:, g_col:g_col + 1] * jax.lax.rsqrt(v + 1e-5)
        cc = auxt[:, h_col:h_col + 1] - m * a
        return jnp.tanh(x * a + cc)

    def put_stats(li, x):
        stats[:, 2 * li:2 * li + 1] += jnp.sum(x, axis=1, keepdims=True)
        stats[:, 2 * li + 1:2 * li + 2] += jnp.sum(x * x, axis=1,
                                                   keepdims=True)

    @pl.when(p == 0)
    def _():
        hdt = _unpack(gdst[...] - gsrc[...])
        x1 = _mm(w0[...], hdt[0:4, :]) + auxt[:, 0:1]
        act[:, blk] = x1
        put_stats(0, x1)

    @pl.when(p == 1)
    def _():
        t1 = bn_tanh(act[:, blk], 0, 1, 2)
        x2 = _mm(w1[...], t1) + auxt[:, 3:4]
        act[:, blk] = x2
        put_stats(1, x2)

    @pl.when(p == 2)
    def _():
        t2 = bn_tanh(act[:, blk], 1, 4, 5)
        x3 = _mm(w2[...], t2) + auxt[:, 6:7]
        act[:, blk] = x3
        put_stats(2, x3)

    @pl.when(p == 3)
    def _():
        t3 = bn_tanh(act[:, blk], 2, 7, 8)
        pay4 = _mm(w3[...], t3) + auxt[0:4, 9:10]
        hdt = _unpack(gdst[...] - gsrc[...])
        mask = jnp.all(hdt == 0.0, axis=0, keepdims=True)
        pay4 = jnp.where(mask, 0.0, pay4)
        pay = jnp.concatenate(
            [pay4, jnp.ones((1, BLK), jnp.float32),
             jnp.zeros((HPAD - LOUT - 1, BLK), jnp.float32)], axis=0)
        out_ref[...] = _pack(pay)


@jax.jit
def _tc_edge_mlp(gdst_p, gsrc_p, w0, w1, w2, w3, auxt):
    edge_map = lambda p, i: (jnp.where((p == 0) | (p == 3), i, 0), 0)
    rep = lambda p, i: (0, 0)
    return pl.pallas_call(
        _edge_mlp_body,
        grid=(4, NBLK),
        in_specs=[
            pl.BlockSpec((PB, 128), edge_map),
            pl.BlockSpec((PB, 128), edge_map),
            pl.BlockSpec((4, EMB), rep),
            pl.BlockSpec((EMB, EMB), rep),
            pl.BlockSpec((EMB, EMB), rep),
            pl.BlockSpec((EMB, LOUT), rep),
            pl.BlockSpec((EMB, 10), rep),
        ],
        out_specs=pl.BlockSpec((PB, 128),
                               lambda p, i: (jnp.where(p == 3, i, 0), 0)),
        out_shape=jax.ShapeDtypeStruct((EPB, 128), jnp.float32),
        scratch_shapes=[
            pltpu.VMEM((EMB, E), jnp.float32),
            pltpu.VMEM((EMB, 8), jnp.float32),
        ],
    )(gdst_p, gsrc_p, w0, w1, w2, w3, auxt)


NPB = N * HPAD // 128


def _node_mlp_body(parts, wu0, wu1, wu2, wu3, auxu, tailt, wp, out_ref):
    def unpack_n(xp):
        t = jnp.transpose(xp)
        return jnp.concatenate(
            [t[HPAD * g:HPAD * (g + 1), :] for g in range(G)], axis=1)

    s = unpack_n(parts[0]) + unpack_n(parts[1])
    cnt = jnp.maximum(s[LOUT:LOUT + 1, :], 1.0)
    aggr = jnp.concatenate([s[0:2, :], s[2:4, :] / cnt], axis=0)

    def bn_tanh(x, g, h):
        m = jnp.mean(x, axis=1, keepdims=True)
        v = jnp.mean((x - m) * (x - m), axis=1, keepdims=True)
        a = g * jax.lax.rsqrt(v + 1e-5)
        return jnp.tanh(x * a + (h - m * a))

    x = _mm(wu0[...], aggr) + auxu[:, 0:1]
    x = bn_tanh(x, auxu[:, 1:2], auxu[:, 2:3])
    x = _mm(wu1[...], x) + auxu[:, 3:4]
    x = bn_tanh(x, auxu[:, 4:5], auxu[:, 5:6])
    x = _mm(wu2[...], x) + auxu[:, 6:7]
    x = bn_tanh(x, auxu[:, 7:8], auxu[:, 8:9])
    x = _mm(wu3[...], x) + tailt[:, 0:1]
    x = bn_tanh(x, tailt[:, 1:2], tailt[:, 2:3])
    o = _mm(wp[...], x) + tailt[0:ODIM, 3:4]
    o16 = jnp.concatenate(
        [o, jnp.zeros((HPAD - ODIM, N), jnp.float32)], axis=0)
    t = jnp.concatenate(
        [o16[:, NPB * g:NPB * (g + 1)] for g in range(G)], axis=0)
    out_ref[...] = jnp.transpose(t)


@jax.jit
def _tc_node_mlp(parts, wu0, wu1, wu2, wu3, auxu, tailt, wp):
    return pl.pallas_call(
        _node_mlp_body,
        out_shape=jax.ShapeDtypeStruct((NPB, 128), jnp.float32),
    )(parts, wu0, wu1, wu2, wu3, auxu, tailt, wp)


def kernel(pos, vel, edge_index, params):
    f32 = jnp.float32
    h16 = jnp.concatenate(
        [pos, vel, jnp.zeros((N, HPAD - 4), f32)], axis=1)
    src = edge_index[0]
    dst = edge_index[1]

    auxt = jnp.stack([
        params['bm0'], params['gm1'], params['hm1'],
        params['bm1'], params['gm2'], params['hm2'],
        params['bm2'], params['gm3'], params['hm3'],
        jnp.concatenate([params['bm3'], jnp.zeros((EMB - LOUT,), f32)]),
    ], axis=1)

    auxu = jnp.stack([
        params['bu0'], params['gu1'], params['hu1'],
        params['bu1'], params['gu2'], params['hu2'],
        params['bu2'], params['gu3'], params['hu3'],
    ], axis=1)
    tailt = jnp.stack([
        params['bu3'], params['gu4'], params['hu4'],
        jnp.concatenate([params['bp'], jnp.zeros((LOUT - ODIM,), f32)]),
    ], axis=1)

    gdst, gsrc = _sc_gather(h16, src, dst)
    payload_p = _tc_edge_mlp(gdst.reshape(EPB, 128),
                             gsrc.reshape(EPB, 128),
                             params['Wm0'], params['Wm1'],
                             params['Wm2'], params['Wm3'], auxt)
    zrows = jnp.zeros((RPS, HPAD), f32)
    parts = _sc_scatter(payload_p.reshape(E, HPAD), dst, zrows)
    out = _tc_node_mlp(parts.reshape(NC, NPB, 128), params['Wu0'],
                       params['Wu1'], params['Wu2'], params['Wu3'],
                       auxu, tailt, params['Wp'])
    return out.reshape(N, HPAD)[:, :ODIM]


# speedup vs baseline: 1.5729x; 1.0868x over previous
import functools

import jax
import jax.numpy as jnp
from jax import lax
from jax.experimental import pallas as pl
from jax.experimental.pallas import tpu as pltpu
from jax.experimental.pallas import tpu_sc as plsc

N = 10000
E = 160000
EMB = 64
LOUT = 4
ODIM = 2
HPAD = 8

NC = 2
NS = 16
NW = NC * NS
CH = E // NW
CHG = 1000
RPS = N // NS

BLK = 32000
NBLK = E // BLK
G = 128 // HPAD
PB = BLK * HPAD // 128
EPB = E * HPAD // 128

def _sc_mesh():
    return plsc.VectorSubcoreMesh(core_axis_name="c", subcore_axis_name="s",
                                  num_cores=NC, num_subcores=NS)


@jax.jit
def _sc_gather(h16, src, dst):
    @functools.partial(
        pl.kernel,
        out_type=(jax.ShapeDtypeStruct((E, HPAD), jnp.float32),
                  jax.ShapeDtypeStruct((E, HPAD), jnp.float32)),
        mesh=_sc_mesh(),
        compiler_params=pltpu.CompilerParams(use_tc_tiling_on_sc=False),
        scratch_types=[
            pltpu.VMEM((CH // CHG, CHG), jnp.int32),
            pltpu.VMEM((CH // CHG, CHG), jnp.int32),
            pltpu.VMEM((CHG, HPAD), jnp.float32),
            pltpu.VMEM((CHG, HPAD), jnp.float32),
            pltpu.SemaphoreType.DMA,
            pltpu.SemaphoreType.DMA,
            pltpu.SemaphoreType.DMA,
        ],
    )
    def k(h_hbm, src_hbm, dst_hbm, gdst_hbm, gsrc_hbm,
          idx_d, idx_s, buf0, buf1, sem_i, sem_g, sem_w):
        wid = lax.axis_index("s") * NC + lax.axis_index("c")
        base = wid * CH
        cpi_d = pltpu.async_copy(dst_hbm.at[wid], idx_d, sem_i)
        cpi_s = pltpu.async_copy(src_hbm.at[wid], idx_s, sem_i)
        cpi_d.wait()
        cpi_s.wait()

        nch = CH // CHG
        bufs = (buf0, buf1)
        outs = []
        for j in range(2 * nch):
            idx = idx_d if j < nch else idx_s
            ohbm = gdst_hbm if j < nch else gsrc_hbm
            c = (j % nch) * CHG
            buf = bufs[j % 2]
            if j >= 2:
                outs[j - 2].wait()
            pltpu.async_copy(h_hbm.at[idx.at[j % nch]], buf, sem_g).wait()
            outs.append(
                pltpu.async_copy(buf, ohbm.at[pl.ds(base + c, CHG)], sem_w))
        outs[-2].wait()
        outs[-1].wait()

    return k(h16, src.reshape(NW, CH // CHG, CHG),
             dst.reshape(NW, CH // CHG, CHG))


@jax.jit
def _sc_scatter(payload, dst, zrows):
    @functools.partial(
        pl.kernel,
        out_type=jax.ShapeDtypeStruct((NC, N, HPAD), jnp.float32),
        mesh=_sc_mesh(),
        compiler_params=pltpu.CompilerParams(use_tc_tiling_on_sc=False),
        scratch_types=[
            pltpu.VMEM((CH // CHG, CHG), jnp.int32),
            pltpu.VMEM((CHG, HPAD), jnp.float32),
            pltpu.VMEM((CHG, HPAD), jnp.float32),
            pltpu.VMEM_SHARED((N, HPAD), jnp.float32),
            pltpu.SemaphoreType.DMA,
            pltpu.SemaphoreType.DMA,
        ],
    )
    def k(pay_hbm, dst_hbm, z_hbm, out_hbm, idx_v, buf0, buf1, acc,
          sem_i, sem_p):
        c = lax.axis_index("c")
        s = lax.axis_index("s")
        wid = s * NC + c
        base = wid * CH
        cpz = pltpu.async_copy(z_hbm, acc.at[pl.ds(s * RPS, RPS)], sem_i)
        cpi = pltpu.async_copy(dst_hbm.at[wid], idx_v, sem_i)
        nch = CH // CHG
        bufs = (buf0, buf1)
        loads = [pltpu.async_copy(pay_hbm.at[pl.ds(base + j * CHG, CHG)],
                                  bufs[j % 2], sem_p) for j in range(2)]
        cpz.wait()
        cpi.wait()
        plsc.subcore_barrier()
        for j in range(nch):
            loads[j].wait()
            pltpu.sync_copy(bufs[j % 2], acc.at[idx_v.at[j]], add=True)
            if j + 2 < nch:
                loads.append(
                    pltpu.async_copy(pay_hbm.at[pl.ds(base + (j + 2) * CHG,
                                                      CHG)],
                                     bufs[j % 2], sem_p))
        plsc.subcore_barrier()
        pltpu.sync_copy(acc.at[pl.ds(s * RPS, RPS)],
                        out_hbm.at[c].at[pl.ds(s * RPS, RPS)])

    return k(payload, dst.reshape(NW, CH // CHG, CHG), zrows)


def _unpack(xp):
    t = jnp.transpose(xp)
    return jnp.concatenate(
        [t[HPAD * g:HPAD * (g + 1), :] for g in range(G)], axis=1)


def _pack(x):
    t = jnp.concatenate(
        [x[:, PB * g:PB * (g + 1)] for g in range(G)], axis=0)
    return jnp.transpose(t)


def _mm(w, x):
    return lax.dot_general(w, x, (((0,), (0,)), ((), ())),
                           preferred_element_type=jnp.float32)


def _edge_mlp_body(gdst, gsrc, w0, w1, w2, w3, auxt, out_ref,
                   act, stats):
    p = pl.program_id(0)
    i = pl.program_id(1)
    blk = pl.ds(i * BLK, BLK)
    inv_e = 1.0 / E

    @pl.when((p == 0) & (i == 0))
    def _():
        stats[...] = jnp.zeros_like(stats)

    def bn_tanh(x, li, g_col, h_col):
        m = stats[:, 2 * li:2 * li + 1] * inv_e
        v = stats[:, 2 * li + 1:2 * li + 2] * inv_e - m * m
        a = auxt[:, g_col:g_col + 1] * jax.lax.rsqrt(v + 1e-5)
        cc = auxt[:, h_col:h_col + 1] - m * a
        return jnp.tanh(x * a + cc)

    def put_stats(li, x):
        stats[:, 2 * li:2 * li + 1] += jnp.sum(x, axis=1, keepdims=True)
        stats[:, 2 * li + 1:2 * li + 2] += jnp.sum(x * x, axis=1,
                                                   keepdims=True)

    @pl.when(p == 0)
    def _():
        hdt = _unpack(gdst[...] - gsrc[...])
        x1 = _mm(w0[...], hdt[0:4, :]) + auxt[:, 0:1]
        act[:, blk] = x1
        put_stats(0, x1)

    @pl.when(p == 1)
    def _():
        t1 = bn_tanh(act[:, blk], 0, 1, 2)
        x2 = _mm(w1[...], t1) + auxt[:, 3:4]
        act[:, blk] = x2
        put_stats(1, x2)

    @pl.when(p == 2)
    def _():
        t2 = bn_tanh(act[:, blk], 1, 4, 5)
        x3 = _mm(w2[...], t2) + auxt[:, 6:7]
        act[:, blk] = x3
        put_stats(2, x3)

    @pl.when(p == 3)
    def _():
        t3 = bn_tanh(act[:, blk], 2, 7, 8)
        pay4 = _mm(w3[...], t3) + auxt[0:4, 9:10]
        hdt = _unpack(gdst[...] - gsrc[...])
        mask = jnp.all(hdt == 0.0, axis=0, keepdims=True)
        pay4 = jnp.where(mask, 0.0, pay4)
        pay = jnp.concatenate(
            [pay4, jnp.ones((1, BLK), jnp.float32),
             jnp.zeros((HPAD - LOUT - 1, BLK), jnp.float32)], axis=0)
        out_ref[...] = _pack(pay)


@jax.jit
def _tc_edge_mlp(gdst_p, gsrc_p, w0, w1, w2, w3, auxt):
    edge_map = lambda p, i: (jnp.where((p == 0) | (p == 3), i, 0), 0)
    rep = lambda p, i: (0, 0)
    return pl.pallas_call(
        _edge_mlp_body,
        grid=(4, NBLK),
        in_specs=[
            pl.BlockSpec((PB, 128), edge_map),
            pl.BlockSpec((PB, 128), edge_map),
            pl.BlockSpec((4, EMB), rep),
            pl.BlockSpec((EMB, EMB), rep),
            pl.BlockSpec((EMB, EMB), rep),
            pl.BlockSpec((EMB, LOUT), rep),
            pl.BlockSpec((EMB, 10), rep),
        ],
        out_specs=pl.BlockSpec((PB, 128),
                               lambda p, i: (jnp.where(p == 3, i, 0), 0)),
        out_shape=jax.ShapeDtypeStruct((EPB, 128), jnp.float32),
        scratch_shapes=[
            pltpu.VMEM((EMB, E), jnp.float32),
            pltpu.VMEM((EMB, 8), jnp.float32),
        ],
    )(gdst_p, gsrc_p, w0, w1, w2, w3, auxt)


NPB = N * HPAD // 128


def _node_mlp_body(parts, wu0, wu1, wu2, wu3, auxu, tailt, wp, out_ref):
    def unpack_n(xp):
        t = jnp.transpose(xp)
        return jnp.concatenate(
            [t[HPAD * g:HPAD * (g + 1), :] for g in range(G)], axis=1)

    s = unpack_n(parts[0]) + unpack_n(parts[1])
    cnt = jnp.maximum(s[LOUT:LOUT + 1, :], 1.0)
    aggr = jnp.concatenate([s[0:2, :], s[2:4, :] / cnt], axis=0)

    def bn_tanh(x, g, h):
        m = jnp.mean(x, axis=1, keepdims=True)
        v = jnp.mean((x - m) * (x - m), axis=1, keepdims=True)
        a = g * jax.lax.rsqrt(v + 1e-5)
        return jnp.tanh(x * a + (h - m * a))

    x = _mm(wu0[...], aggr) + auxu[:, 0:1]
    x = bn_tanh(x, auxu[:, 1:2], auxu[:, 2:3])
    x = _mm(wu1[...], x) + auxu[:, 3:4]
    x = bn_tanh(x, auxu[:, 4:5], auxu[:, 5:6])
    x = _mm(wu2[...], x) + auxu[:, 6:7]
    x = bn_tanh(x, auxu[:, 7:8], auxu[:, 8:9])
    x = _mm(wu3[...], x) + tailt[:, 0:1]
    x = bn_tanh(x, tailt[:, 1:2], tailt[:, 2:3])
    o = _mm(wp[...], x) + tailt[0:ODIM, 3:4]
    o16 = jnp.concatenate(
        [o, jnp.zeros((HPAD - ODIM, N), jnp.float32)], axis=0)
    t = jnp.concatenate(
        [o16[:, NPB * g:NPB * (g + 1)] for g in range(G)], axis=0)
    out_ref[...] = jnp.transpose(t)


@jax.jit
def _tc_node_mlp(parts, wu0, wu1, wu2, wu3, auxu, tailt, wp):
    return pl.pallas_call(
        _node_mlp_body,
        out_shape=jax.ShapeDtypeStruct((NPB, 128), jnp.float32),
    )(parts, wu0, wu1, wu2, wu3, auxu, tailt, wp)


def kernel(pos, vel, edge_index, params):
    f32 = jnp.float32
    h16 = jnp.concatenate(
        [pos, vel, jnp.zeros((N, HPAD - 4), f32)], axis=1)
    src = edge_index[0]
    dst = edge_index[1]

    auxt = jnp.stack([
        params['bm0'], params['gm1'], params['hm1'],
        params['bm1'], params['gm2'], params['hm2'],
        params['bm2'], params['gm3'], params['hm3'],
        jnp.concatenate([params['bm3'], jnp.zeros((EMB - LOUT,), f32)]),
    ], axis=1)

    auxu = jnp.stack([
        params['bu0'], params['gu1'], params['hu1'],
        params['bu1'], params['gu2'], params['hu2'],
        params['bu2'], params['gu3'], params['hu3'],
    ], axis=1)
    tailt = jnp.stack([
        params['bu3'], params['gu4'], params['hu4'],
        jnp.concatenate([params['bp'], jnp.zeros((LOUT - ODIM,), f32)]),
    ], axis=1)

    gdst, gsrc = _sc_gather(h16, src, dst)
    payload_p = _tc_edge_mlp(gdst.reshape(EPB, 128),
                             gsrc.reshape(EPB, 128),
                             params['Wm0'], params['Wm1'],
                             params['Wm2'], params['Wm3'], auxt)
    zrows = jnp.zeros((RPS, HPAD), f32)
    parts = _sc_scatter(payload_p.reshape(E, HPAD), dst, zrows)
    out = _tc_node_mlp(parts.reshape(NC, NPB, 128), params['Wu0'],
                       params['Wu1'], params['Wu2'], params['Wu3'],
                       auxu, tailt, params['Wp'])
    return out.reshape(N, HPAD)[:, :ODIM]


# speedup vs baseline: 1.5894x; 1.0105x over previous
import functools

import jax
import jax.numpy as jnp
from jax import lax
from jax.experimental import pallas as pl
from jax.experimental.pallas import tpu as pltpu
from jax.experimental.pallas import tpu_sc as plsc

N = 10000
E = 160000
EMB = 64
LOUT = 4
ODIM = 2
HPAD = 8

NC = 2
NS = 16
NW = NC * NS
CH = E // NW
CHG = 1000
RPS = N // NS

BLK = 32000
NBLK = E // BLK
G = 128 // HPAD
PB = BLK * HPAD // 128
EPB = E * HPAD // 128

def _sc_mesh():
    return plsc.VectorSubcoreMesh(core_axis_name="c", subcore_axis_name="s",
                                  num_cores=NC, num_subcores=NS)


@jax.jit
def _sc_gather(h16, src, dst):
    @functools.partial(
        pl.kernel,
        out_type=(jax.ShapeDtypeStruct((E, HPAD), jnp.float32),
                  jax.ShapeDtypeStruct((E, HPAD), jnp.float32)),
        mesh=_sc_mesh(),
        compiler_params=pltpu.CompilerParams(use_tc_tiling_on_sc=False),
        scratch_types=[
            pltpu.VMEM((CH // CHG, CHG), jnp.int32),
            pltpu.VMEM((CH // CHG, CHG), jnp.int32),
            pltpu.VMEM((CHG, HPAD), jnp.float32),
            pltpu.VMEM((CHG, HPAD), jnp.float32),
            pltpu.VMEM((CHG, HPAD), jnp.float32),
            pltpu.VMEM((CHG, HPAD), jnp.float32),
            pltpu.SemaphoreType.DMA,
            [pltpu.SemaphoreType.DMA] * 4,
            [pltpu.SemaphoreType.DMA] * 4,
        ],
    )
    def k(h_hbm, src_hbm, dst_hbm, gdst_hbm, gsrc_hbm,
          idx_d, idx_s, b0, b1, b2, b3, sem_i, sem_g, sem_w):
        wid = lax.axis_index("s") * NC + lax.axis_index("c")
        base = wid * CH
        cpi_d = pltpu.async_copy(dst_hbm.at[wid], idx_d, sem_i)
        cpi_s = pltpu.async_copy(src_hbm.at[wid], idx_s, sem_i)
        cpi_d.wait()
        cpi_s.wait()

        nch = CH // CHG
        nit = 2 * nch
        bufs = (b0, b1, b2, b3)

        def idx_of(j):
            return (idx_d if j < nch else idx_s).at[j % nch]

        def out_of(j):
            ohbm = gdst_hbm if j < nch else gsrc_hbm
            return ohbm.at[pl.ds(base + (j % nch) * CHG, CHG)]

        gath = [pltpu.async_copy(h_hbm.at[idx_of(j)], bufs[j % 4],
                                 sem_g[j % 4]) for j in range(2)]
        wr = []
        for j in range(nit):
            gath[j].wait()
            wr.append(pltpu.async_copy(bufs[j % 4], out_of(j), sem_w[j % 4]))
            if j + 2 < nit:
                if j >= 2:
                    wr[j - 2].wait()
                gath.append(
                    pltpu.async_copy(h_hbm.at[idx_of(j + 2)],
                                     bufs[(j + 2) % 4], sem_g[(j + 2) % 4]))
        for j in range(max(0, nit - 4), nit):
            wr[j].wait()

    return k(h16, src.reshape(NW, CH // CHG, CHG),
             dst.reshape(NW, CH // CHG, CHG))


@jax.jit
def _sc_scatter(payload, dst, zrows):
    @functools.partial(
        pl.kernel,
        out_type=jax.ShapeDtypeStruct((NC, N, HPAD), jnp.float32),
        mesh=_sc_mesh(),
        compiler_params=pltpu.CompilerParams(use_tc_tiling_on_sc=False),
        scratch_types=[
            pltpu.VMEM((CH // CHG, CHG), jnp.int32),
            pltpu.VMEM((CHG, HPAD), jnp.float32),
            pltpu.VMEM((CHG, HPAD), jnp.float32),
            pltpu.VMEM_SHARED((N, HPAD), jnp.float32),
            pltpu.SemaphoreType.DMA,
            pltpu.SemaphoreType.DMA,
        ],
    )
    def k(pay_hbm, dst_hbm, z_hbm, out_hbm, idx_v, buf0, buf1, acc,
          sem_i, sem_p):
        c = lax.axis_index("c")
        s = lax.axis_index("s")
        wid = s * NC + c
        base = wid * CH
        cpz = pltpu.async_copy(z_hbm, acc.at[pl.ds(s * RPS, RPS)], sem_i)
        cpi = pltpu.async_copy(dst_hbm.at[wid], idx_v, sem_i)
        nch = CH // CHG
        bufs = (buf0, buf1)
        loads = [pltpu.async_copy(pay_hbm.at[pl.ds(base + j * CHG, CHG)],
                                  bufs[j % 2], sem_p) for j in range(2)]
        cpz.wait()
        cpi.wait()
        plsc.subcore_barrier()
        for j in range(nch):
            loads[j].wait()
            pltpu.sync_copy(bufs[j % 2], acc.at[idx_v.at[j]], add=True)
            if j + 2 < nch:
                loads.append(
                    pltpu.async_copy(pay_hbm.at[pl.ds(base + (j + 2) * CHG,
                                                      CHG)],
                                     bufs[j % 2], sem_p))
        plsc.subcore_barrier()
        pltpu.sync_copy(acc.at[pl.ds(s * RPS, RPS)],
                        out_hbm.at[c].at[pl.ds(s * RPS, RPS)])

    return k(payload, dst.reshape(NW, CH // CHG, CHG), zrows)


def _unpack(xp):
    t = jnp.transpose(xp)
    return jnp.concatenate(
        [t[HPAD * g:HPAD * (g + 1), :] for g in range(G)], axis=1)


def _pack(x):
    t = jnp.concatenate(
        [x[:, PB * g:PB * (g + 1)] for g in range(G)], axis=0)
    return jnp.transpose(t)


def _mm(w, x):
    return lax.dot_general(w, x, (((0,), (0,)), ((), ())),
                           preferred_element_type=jnp.float32)


def _edge_mlp_body(gdst, gsrc, w0, w1, w2, w3, auxt, out_ref,
                   act, stats):
    p = pl.program_id(0)
    i = pl.program_id(1)
    blk = pl.ds(i * BLK, BLK)
    inv_e = 1.0 / E

    @pl.when((p == 0) & (i == 0))
    def _():
        stats[...] = jnp.zeros_like(stats)

    def bn_tanh(x, li, g_col, h_col):
        m = stats[:, 2 * li:2 * li + 1] * inv_e
        v = stats[:, 2 * li + 1:2 * li + 2] * inv_e - m * m
        a = auxt[:, g_col:g_col + 1] * jax.lax.rsqrt(v + 1e-5)
        cc = auxt[:, h_col:h_col + 1] - m * a
        return jnp.tanh(x * a + cc)

    def put_stats(li, x):
        stats[:, 2 * li:2 * li + 1] += jnp.sum(x, axis=1, keepdims=True)
        stats[:, 2 * li + 1:2 * li + 2] += jnp.sum(x * x, axis=1,
                                                   keepdims=True)

    @pl.when(p == 0)
    def _():
        hdt = _unpack(gdst[...] - gsrc[...])
        x1 = _mm(w0[...], hdt[0:4, :]) + auxt[:, 0:1]
        act[:, blk] = x1
        put_stats(0, x1)

    @pl.when(p == 1)
    def _():
        t1 = bn_tanh(act[:, blk], 0, 1, 2)
        x2 = _mm(w1[...], t1) + auxt[:, 3:4]
        act[:, blk] = x2
        put_stats(1, x2)

    @pl.when(p == 2)
    def _():
        t2 = bn_tanh(act[:, blk], 1, 4, 5)
        x3 = _mm(w2[...], t2) + auxt[:, 6:7]
        act[:, blk] = x3
        put_stats(2, x3)

    @pl.when(p == 3)
    def _():
        t3 = bn_tanh(act[:, blk], 2, 7, 8)
        pay4 = _mm(w3[...], t3) + auxt[0:4, 9:10]
        hdt = _unpack(gdst[...] - gsrc[...])
        mask = jnp.all(hdt == 0.0, axis=0, keepdims=True)
        pay4 = jnp.where(mask, 0.0, pay4)
        pay = jnp.concatenate(
            [pay4, jnp.ones((1, BLK), jnp.float32),
             jnp.zeros((HPAD - LOUT - 1, BLK), jnp.float32)], axis=0)
        out_ref[...] = _pack(pay)


@jax.jit
def _tc_edge_mlp(gdst_p, gsrc_p, w0, w1, w2, w3, auxt):
    edge_map = lambda p, i: (jnp.where((p == 0) | (p == 3), i, 0), 0)
    rep = lambda p, i: (0, 0)
    return pl.pallas_call(
        _edge_mlp_body,
        grid=(4, NBLK),
        in_specs=[
            pl.BlockSpec((PB, 128), edge_map),
            pl.BlockSpec((PB, 128), edge_map),
            pl.BlockSpec((4, EMB), rep),
            pl.BlockSpec((EMB, EMB), rep),
            pl.BlockSpec((EMB, EMB), rep),
            pl.BlockSpec((EMB, LOUT), rep),
            pl.BlockSpec((EMB, 10), rep),
        ],
        out_specs=pl.BlockSpec((PB, 128),
                               lambda p, i: (jnp.where(p == 3, i, 0), 0)),
        out_shape=jax.ShapeDtypeStruct((EPB, 128), jnp.float32),
        scratch_shapes=[
            pltpu.VMEM((EMB, E), jnp.float32),
            pltpu.VMEM((EMB, 8), jnp.float32),
        ],
    )(gdst_p, gsrc_p, w0, w1, w2, w3, auxt)


NPB = N * HPAD // 128


def _node_mlp_body(parts, wu0, wu1, wu2, wu3, auxu, tailt, wp, out_ref):
    def unpack_n(xp):
        t = jnp.transpose(xp)
        return jnp.concatenate(
            [t[HPAD * g:HPAD * (g + 1), :] for g in range(G)], axis=1)

    s = unpack_n(parts[0]) + unpack_n(parts[1])
    cnt = jnp.maximum(s[LOUT:LOUT + 1, :], 1.0)
    aggr = jnp.concatenate([s[0:2, :], s[2:4, :] / cnt], axis=0)

    def bn_tanh(x, g, h):
        m = jnp.mean(x, axis=1, keepdims=True)
        v = jnp.mean((x - m) * (x - m), axis=1, keepdims=True)
        a = g * jax.lax.rsqrt(v + 1e-5)
        return jnp.tanh(x * a + (h - m * a))

    x = _mm(wu0[...], aggr) + auxu[:, 0:1]
    x = bn_tanh(x, auxu[:, 1:2], auxu[:, 2:3])
    x = _mm(wu1[...], x) + auxu[:, 3:4]
    x = bn_tanh(x, auxu[:, 4:5], auxu[:, 5:6])
    x = _mm(wu2[...], x) + auxu[:, 6:7]
    x = bn_tanh(x, auxu[:, 7:8], auxu[:, 8:9])
    x = _mm(wu3[...], x) + tailt[:, 0:1]
    x = bn_tanh(x, tailt[:, 1:2], tailt[:, 2:3])
    o = _mm(wp[...], x) + tailt[0:ODIM, 3:4]
    o16 = jnp.concatenate(
        [o, jnp.zeros((HPAD - ODIM, N), jnp.float32)], axis=0)
    t = jnp.concatenate(
        [o16[:, NPB * g:NPB * (g + 1)] for g in range(G)], axis=0)
    out_ref[...] = jnp.transpose(t)


@jax.jit
def _tc_node_mlp(parts, wu0, wu1, wu2, wu3, auxu, tailt, wp):
    return pl.pallas_call(
        _node_mlp_body,
        out_shape=jax.ShapeDtypeStruct((NPB, 128), jnp.float32),
    )(parts, wu0, wu1, wu2, wu3, auxu, tailt, wp)


def kernel(pos, vel, edge_index, params):
    f32 = jnp.float32
    h16 = jnp.concatenate(
        [pos, vel, jnp.zeros((N, HPAD - 4), f32)], axis=1)
    src = edge_index[0]
    dst = edge_index[1]

    auxt = jnp.stack([
        params['bm0'], params['gm1'], params['hm1'],
        params['bm1'], params['gm2'], params['hm2'],
        params['bm2'], params['gm3'], params['hm3'],
        jnp.concatenate([params['bm3'], jnp.zeros((EMB - LOUT,), f32)]),
    ], axis=1)

    auxu = jnp.stack([
        params['bu0'], params['gu1'], params['hu1'],
        params['bu1'], params['gu2'], params['hu2'],
        params['bu2'], params['gu3'], params['hu3'],
    ], axis=1)
    tailt = jnp.stack([
        params['bu3'], params['gu4'], params['hu4'],
        jnp.concatenate([params['bp'], jnp.zeros((LOUT - ODIM,), f32)]),
    ], axis=1)

    gdst, gsrc = _sc_gather(h16, src, dst)
    payload_p = _tc_edge_mlp(gdst.reshape(EPB, 128),
                             gsrc.reshape(EPB, 128),
                             params['Wm0'], params['Wm1'],
                             params['Wm2'], params['Wm3'], auxt)
    zrows = jnp.zeros((RPS, HPAD), f32)
    parts = _sc_scatter(payload_p.reshape(E, HPAD), dst, zrows)
    out = _tc_node_mlp(parts.reshape(NC, NPB, 128), params['Wu0'],
                       params['Wu1'], params['Wu2'], params['Wu3'],
                       auxu, tailt, params['Wp'])
    return out.reshape(N, HPAD)[:, :ODIM]
